# Initial kernel scaffold; baseline (speedup 1.0000x reference)
import jax
import jax.numpy as jnp
from jax.experimental import pallas as pl


def kernel(x, pos, batch, query_pos, query_pos_batch, params):
    raise NotImplementedError("write your pallas kernel here")


# speedup vs baseline: 4.1390x; 4.1390x over previous
import functools

import jax
import jax.numpy as jnp
from jax.experimental import pallas as pl
from jax.experimental.pallas import tpu as pltpu

_B = 4
_KNN = 16
_F32 = jnp.float32


_HI = jax.lax.Precision.HIGHEST


def _dot(a, b):
    return jnp.dot(a, b, preferred_element_type=_F32, precision=_HI)


def _relu(v):
    return jnp.maximum(v, 0.0)


def _fps_body(m, n, coords_ref, out_ref):
    cx = coords_ref[0]
    cy = coords_ref[1]
    cz = coords_ref[2]
    iota = jax.lax.broadcasted_iota(jnp.int32, (_B, n), 1)
    x0 = cx[:, 0:1]
    y0 = cy[:, 0:1]
    z0 = cz[:, 0:1]
    d0 = (cx - x0) ** 2 + (cy - y0) ** 2 + (cz - z0) ** 2
    for b in range(_B):
        out_ref[b, 0] = 0
    iota1 = jax.lax.broadcasted_iota(jnp.int32, (1, n), 1)

    def body(i, d):
        nxts = []
        for b in range(_B):
            db = d[b:b + 1, :]
            rmax = jnp.max(db)
            nxt = jnp.min(jnp.where(db == rmax, iota1, n))
            out_ref[b, i] = nxt
            nxts.append(nxt)
        nxtcol = jnp.concatenate(
            [jnp.full((1, 1), s, jnp.int32) for s in nxts], axis=0)
        sel = iota == nxtcol
        xn = jnp.sum(jnp.where(sel, cx, 0.0), axis=1, keepdims=True)
        yn = jnp.sum(jnp.where(sel, cy, 0.0), axis=1, keepdims=True)
        zn = jnp.sum(jnp.where(sel, cz, 0.0), axis=1, keepdims=True)
        dn = (cx - xn) ** 2 + (cy - yn) ** 2 + (cz - zn) ** 2
        return jnp.minimum(d, dn)

    jax.lax.fori_loop(1, m, body, d0)


def _fps(pos_bn3, m):
    n = pos_bn3.shape[1]
    coords = jnp.transpose(pos_bn3, (2, 0, 1))
    return pl.pallas_call(
        functools.partial(_fps_body, m, n),
        out_specs=pl.BlockSpec(memory_space=pltpu.SMEM),
        out_shape=jax.ShapeDtypeStruct((_B, m), jnp.int32),
    )(coords)


def _gather_body(n, idx_ref, src_ref, out_ref):
    idx = idx_ref[0]
    src = src_ref[0]
    tm = idx.shape[0]
    iota = jax.lax.broadcasted_iota(jnp.int32, (tm, n), 1)
    oh = (iota == idx).astype(_F32)
    out_ref[0] = _dot(oh, src)


def _gather_rows(src, idx):
    n, c = src.shape[1], src.shape[2]
    m = idx.shape[1]
    tm = min(m, 512)
    idx3 = idx.reshape(_B, m, 1)
    return pl.pallas_call(
        functools.partial(_gather_body, n),
        grid=(_B, m // tm),
        in_specs=[
            pl.BlockSpec((1, tm, 1), lambda b, t: (b, t, 0)),
            pl.BlockSpec((1, n, c), lambda b, t: (b, 0, 0)),
        ],
        out_specs=pl.BlockSpec((1, tm, c), lambda b, t: (b, t, 0)),
        out_shape=jax.ShapeDtypeStruct((_B, m, c), _F32),
    )(idx3, src)


def _knn_body(k, n, srcc_ref, dst_ref, idx_ref, val_ref):
    sc = srcc_ref[0]
    sx = sc[0:1, :]
    sy = sc[1:2, :]
    sz = sc[2:3, :]
    pd = dst_ref[0]
    dx = pd[:, 0:1]
    dy = pd[:, 1:2]
    dz = pd[:, 2:3]
    d2 = (dx - sx) ** 2 + (dy - sy) ** 2 + (dz - sz) ** 2
    tm = d2.shape[0]
    iota = jax.lax.broadcasted_iota(jnp.int32, (tm, n), 1)
    idx_cols = []
    val_cols = []
    for _ in range(k):
        v = jnp.min(d2, axis=1, keepdims=True)
        ix = jnp.min(jnp.where(d2 == v, iota, n), axis=1, keepdims=True)
        idx_cols.append(ix)
        val_cols.append(v)
        d2 = jnp.where(iota == ix, jnp.inf, d2)
    idx_ref[0] = jnp.concatenate(idx_cols, axis=1)
    val_ref[0] = jnp.concatenate(val_cols, axis=1)


def _knn(pos_src, pos_dst, k):
    n = pos_src.shape[1]
    m = pos_dst.shape[1]
    tm = min(m, 256)
    srcc = jnp.transpose(pos_src, (0, 2, 1))
    return pl.pallas_call(
        functools.partial(_knn_body, k, n),
        grid=(_B, m // tm),
        in_specs=[
            pl.BlockSpec((1, 3, n), lambda b, t: (b, 0, 0)),
            pl.BlockSpec((1, tm, 3), lambda b, t: (b, t, 0)),
        ],
        out_specs=[
            pl.BlockSpec((1, tm, k), lambda b, t: (b, t, 0)),
            pl.BlockSpec((1, tm, k), lambda b, t: (b, t, 0)),
        ],
        out_shape=[
            jax.ShapeDtypeStruct((_B, m, k), jnp.int32),
            jax.ShapeDtypeStruct((_B, m, k), _F32),
        ],
    )(srcc, pos_dst)


def _sa_body(k, n, has_x, nbr_ref, pd_ref, ps_ref, *rest):
    if has_x:
        (xs_ref, w1_ref, b1_ref, w2_ref, b2_ref, wsc_ref, bsc_ref,
         out_ref) = rest
        src = jnp.concatenate([xs_ref[0], ps_ref[0]], axis=1)
    else:
        w1_ref, b1_ref, w2_ref, b2_ref, wsc_ref, bsc_ref, out_ref = rest
        src = ps_ref[0]
    nbr = nbr_ref[0]
    pd = pd_ref[0]
    tm = nbr.shape[0]
    c = src.shape[1] - 3
    if c:
        pdpad = jnp.concatenate([jnp.zeros((tm, c), _F32), pd], axis=1)
    else:
        pdpad = pd
    iota = jax.lax.broadcasted_iota(jnp.int32, (tm, n), 1)
    w1 = w1_ref[:]
    b1 = b1_ref[:]
    w2 = w2_ref[:]
    b2 = b2_ref[:]
    wsc = wsc_ref[:]
    bsc = bsc_ref[:]
    acc = None
    for j in range(k):
        oh = (iota == nbr[:, j:j + 1]).astype(_F32)
        g = _dot(oh, src)
        h0 = g - pdpad
        h = _dot(_relu(_dot(h0, w1) + b1), w2) + b2
        h = h + _dot(h0, wsc) + bsc
        h = _relu(h)
        acc = h if acc is None else jnp.maximum(acc, h)
    out_ref[0] = acc


def _sa(nbr, pos_dst, pos_src, x_src, w):
    n = pos_src.shape[1]
    m = nbr.shape[1]
    k = nbr.shape[2]
    tm = min(m, 128)
    cout = w['l2']['w'].shape[1]
    has_x = x_src is not None
    cin = w['l1']['w'].shape[0]
    hid = w['l1']['w'].shape[1]
    cx = x_src.shape[2] if has_x else 0
    wspec = lambda s: pl.BlockSpec(s, lambda b, t: tuple(0 for _ in s))
    in_specs = [
        pl.BlockSpec((1, tm, k), lambda b, t: (b, t, 0)),
        pl.BlockSpec((1, tm, 3), lambda b, t: (b, t, 0)),
        pl.BlockSpec((1, n, 3), lambda b, t: (b, 0, 0)),
    ]
    args = [nbr, pos_dst, pos_src]
    if has_x:
        in_specs.append(pl.BlockSpec((1, n, cx), lambda b, t: (b, 0, 0)))
        args.append(x_src)
    in_specs += [wspec((cin, hid)), wspec((1, hid)), wspec((hid, cout)),
                 wspec((1, cout)), wspec((cin, cout)), wspec((1, cout))]
    args += [w['l1']['w'], w['l1']['b'].reshape(1, -1),
             w['l2']['w'], w['l2']['b'].reshape(1, -1),
             w['sc']['w'], w['sc']['b'].reshape(1, -1)]
    return pl.pallas_call(
        functools.partial(_sa_body, k, n, has_x),
        grid=(_B, m // tm),
        in_specs=in_specs,
        out_specs=pl.BlockSpec((1, tm, cout), lambda b, t: (b, t, 0)),
        out_shape=jax.ShapeDtypeStruct((_B, m, cout), _F32),
    )(*args)


def _fp_body(ns, has_xt, has_sc, final, srcc_ref, srcp_ref, xs_ref, *rest):
    rest = list(rest)
    xt_ref = rest.pop(0) if has_xt else None
    w1_ref, b1_ref, w2_ref, b2_ref = rest[:4]
    rest = rest[4:]
    if has_sc:
        wsc_ref, bsc_ref = rest[:2]
        rest = rest[2:]
    if final:
        f1w_ref, f1b_ref, f2w_ref, f2b_ref = rest[:4]
        rest = rest[4:]
    pt_ref = rest[0]
    out_ref = rest[1]

    sc = srcc_ref[0]
    sx = sc[0:1, :]
    sy = sc[1:2, :]
    sz = sc[2:3, :]
    pt = pt_ref[0]
    dx = pt[:, 0:1]
    dy = pt[:, 1:2]
    dz = pt[:, 2:3]
    d2 = (dx - sx) ** 2 + (dy - sy) ** 2 + (dz - sz) ** 2
    tm = d2.shape[0]
    iota = jax.lax.broadcasted_iota(jnp.int32, (tm, ns), 1)
    xs = xs_ref[0]
    num = None
    den = None
    for _ in range(3):
        v = jnp.min(d2, axis=1, keepdims=True)
        ix = jnp.min(jnp.where(d2 == v, iota, ns), axis=1, keepdims=True)
        d2 = jnp.where(iota == ix, jnp.inf, d2)
        oh = (iota == ix).astype(_F32)
        xg = _dot(oh, xs)
        wgt = 1.0 / (v + 1e-16)
        contrib = xg * wgt
        num = contrib if num is None else num + contrib
        den = wgt if den is None else den + wgt
    interp = num / den
    if has_xt:
        comb = jnp.concatenate([xt_ref[0], interp], axis=1)
    else:
        comb = interp
    h = _dot(_relu(_dot(comb, w1_ref[:]) + b1_ref[:]),
             w2_ref[:]) + b2_ref[:]
    if has_sc:
        h = h + _dot(comb, wsc_ref[:]) \
            + bsc_ref[:]
    else:
        h = h + comb
    h = _relu(h)
    if final:
        h = _relu(_dot(h, f1w_ref[:])
                  + f1b_ref[:])
        h = _dot(h, f2w_ref[:]) + f2b_ref[:]
    out_ref[0] = h


def _fp(x_tgt, pos_tgt, x_src, pos_src, w, final=None):
    ns = pos_src.shape[1]
    mt = pos_tgt.shape[1]
    tm = min(mt, 256)
    c = x_src.shape[2]
    has_xt = x_tgt is not None
    has_sc = 'sc' in w
    cin = w['l1']['w'].shape[0]
    hid = w['l1']['w'].shape[1]
    cout = w['l2']['w'].shape[1]
    srcc = jnp.transpose(pos_src, (0, 2, 1))
    wspec = lambda s: pl.BlockSpec(s, lambda b, t: tuple(0 for _ in s))
    in_specs = [
        pl.BlockSpec((1, 3, ns), lambda b, t: (b, 0, 0)),
        pl.BlockSpec((1, ns, 3), lambda b, t: (b, 0, 0)),
        pl.BlockSpec((1, ns, c), lambda b, t: (b, 0, 0)),
    ]
    args = [srcc, pos_src, x_src]
    if has_xt:
        ct = x_tgt.shape[2]
        in_specs.append(pl.BlockSpec((1, tm, ct), lambda b, t: (b, t, 0)))
        args.append(x_tgt)
    in_specs += [wspec((cin, hid)), wspec((1, hid)), wspec((hid, cout)),
                 wspec((1, cout))]
    args += [w['l1']['w'], w['l1']['b'].reshape(1, -1),
             w['l2']['w'], w['l2']['b'].reshape(1, -1)]
    if has_sc:
        in_specs += [wspec((cin, cout)), wspec((1, cout))]
        args += [w['sc']['w'], w['sc']['b'].reshape(1, -1)]
    cfin = cout
    if final is not None:
        f1, f2 = final
        h1 = f1['w'].shape[1]
        cfin = f2['w'].shape[1]
        in_specs += [wspec((cout, h1)), wspec((1, h1)),
                     wspec((h1, cfin)), wspec((1, cfin))]
        args += [f1['w'], f1['b'].reshape(1, -1),
                 f2['w'], f2['b'].reshape(1, -1)]
    in_specs.append(pl.BlockSpec((1, tm, 3), lambda b, t: (b, t, 0)))
    args.append(pos_tgt)
    return pl.pallas_call(
        functools.partial(_fp_body, ns, has_xt, has_sc, final is not None),
        grid=(_B, mt // tm),
        in_specs=in_specs,
        out_specs=pl.BlockSpec((1, tm, cfin), lambda b, t: (b, t, 0)),
        out_shape=jax.ShapeDtypeStruct((_B, mt, cfin), _F32),
    )(*args)


def _head_body(x4_ref, w1_ref, b1_ref, w2_ref, b2_ref, gw_ref, gb_ref,
               out_ref):
    g = jnp.max(x4_ref[0], axis=0, keepdims=True)
    h = _dot(_relu(_dot(g, w1_ref[:]) + b1_ref[:]),
             w2_ref[:]) + b2_ref[:]
    h = _relu(h + g)
    out_ref[0] = _dot(h, gw_ref[:]) \
        + gb_ref[:]


def _head(x4, gm1, gm2):
    mper = x4.shape[1]
    d = x4.shape[2]
    cc = gm2['w'].shape[1]
    wspec = lambda s: pl.BlockSpec(s, lambda b: tuple(0 for _ in s))
    return pl.pallas_call(
        _head_body,
        grid=(_B,),
        in_specs=[
            pl.BlockSpec((1, mper, d), lambda b: (b, 0, 0)),
            wspec((d, d)), wspec((1, d)), wspec((d, d)), wspec((1, d)),
            wspec((d, cc)), wspec((1, cc)),
        ],
        out_specs=pl.BlockSpec((1, 1, cc), lambda b: (b, 0, 0)),
        out_shape=jax.ShapeDtypeStruct((_B, 1, cc), _F32),
    )(x4, gm1['l1']['w'], gm1['l1']['b'].reshape(1, -1),
      gm1['l2']['w'], gm1['l2']['b'].reshape(1, -1),
      gm2['w'], gm2['b'].reshape(1, -1))


def _bott_body(x3_ref, z_ref, c_ref, w1_ref, b1_ref, w2_ref, b2_ref,
               wsc_ref, bsc_ref, out_ref):
    x3 = x3_ref[0]
    m = x3.shape[0]
    zb = jnp.broadcast_to(z_ref[0], (m, z_ref.shape[2]))
    cb = jnp.broadcast_to(c_ref[0], (m, c_ref.shape[2]))
    comb = jnp.concatenate([x3, zb, cb], axis=1)
    h = _dot(_relu(_dot(comb, w1_ref[:]) + b1_ref[:]),
             w2_ref[:]) + b2_ref[:]
    h = h + _dot(comb, wsc_ref[:]) \
        + bsc_ref[:]
    out_ref[0] = _relu(h)


def _bott(x3, z, c, w):
    m = x3.shape[1]
    cx = x3.shape[2]
    zd = z.shape[2]
    cd = c.shape[2]
    cin = w['l1']['w'].shape[0]
    hid = w['l1']['w'].shape[1]
    cout = w['l2']['w'].shape[1]
    wspec = lambda s: pl.BlockSpec(s, lambda b: tuple(0 for _ in s))
    return pl.pallas_call(
        _bott_body,
        grid=(_B,),
        in_specs=[
            pl.BlockSpec((1, m, cx), lambda b: (b, 0, 0)),
            pl.BlockSpec((1, 1, zd), lambda b: (b, 0, 0)),
            pl.BlockSpec((1, 1, cd), lambda b: (b, 0, 0)),
            wspec((cin, hid)), wspec((1, hid)), wspec((hid, cout)),
            wspec((1, cout)), wspec((cin, cout)), wspec((1, cout)),
        ],
        out_specs=pl.BlockSpec((1, m, cout), lambda b: (b, 0, 0)),
        out_shape=jax.ShapeDtypeStruct((_B, m, cout), _F32),
    )(x3, z, c,
      w['l1']['w'], w['l1']['b'].reshape(1, -1),
      w['l2']['w'], w['l2']['b'].reshape(1, -1),
      w['sc']['w'], w['sc']['b'].reshape(1, -1))


def _sa_stage(x_src, pos, m, w):
    samp = _fps(pos, m)
    pos_dst = _gather_rows(pos, samp)
    nbr, _ = _knn(pos, pos_dst, _KNN)
    xo = _sa(nbr, pos_dst, pos, x_src, w)
    return xo, pos_dst


def kernel(x, pos, batch, query_pos, query_pos_batch, params):
    cond = params['cond']
    dec = params['dec']
    pos0 = pos.reshape(_B, -1, 3)
    x0 = x.reshape(_B, -1, 3)

    xe1, pe1 = _sa_stage(x0, pos0, 1024, cond['sa1'])
    xe2, pe2 = _sa_stage(xe1, pe1, 512, cond['sa2'])
    xe3, pe3 = _sa_stage(xe2, pe2, 256, cond['sa3'])
    xe4, _ = _sa_stage(xe3, pe3, 128, cond['sa4'])
    c = _head(xe4, cond['gm1'], cond['gm2'])

    q0 = query_pos.reshape(_B, -1, 3)
    xd1, pd1 = _sa_stage(None, q0, 1024, dec['sa1'])
    xd2, pd2 = _sa_stage(xd1, pd1, 256, dec['sa2'])
    xd3, pd3 = _sa_stage(xd2, pd2, 64, dec['sa3'])

    z = jax.random.normal(jax.random.key(42), (_B, 64), dtype=_F32)
    bott = _bott(xd3, z.reshape(_B, 1, 64), c, dec['bott'])

    up3 = _fp(xd3, pd3, bott, pd3, dec['fp1'])
    up2 = _fp(xd2, pd2, up3, pd3, dec['fp2'])
    up1 = _fp(xd1, pd1, up2, pd2, dec['fp3'])
    out = _fp(None, q0, up1, pd1, dec['fp4'],
              final=(dec['f1'], dec['f2']))
    return out.reshape(-1, 3)


# speedup vs baseline: 7.5674x; 1.8283x over previous
import functools

import jax
import jax.numpy as jnp
from jax.experimental import pallas as pl
from jax.experimental.pallas import tpu as pltpu

_B = 4
_KNN = 16
_F32 = jnp.float32


_HI = jax.lax.Precision.HIGHEST


def _dot(a, b):
    return jnp.dot(a, b, preferred_element_type=_F32, precision=_HI)


def _relu(v):
    return jnp.maximum(v, 0.0)


_BF = jnp.bfloat16


def _split3(src):
    hi = src.astype(_BF)
    r1 = src - hi.astype(_F32)
    mid = r1.astype(_BF)
    lo = (r1 - mid.astype(_F32)).astype(_BF)
    return hi, mid, lo


def _oh_gather(oh_bf, parts):
    hi, mid, lo = parts
    d = jnp.dot(oh_bf, hi, preferred_element_type=_F32)
    d = d + jnp.dot(oh_bf, mid, preferred_element_type=_F32)
    return d + jnp.dot(oh_bf, lo, preferred_element_type=_F32)


def _fps_prep(c_ref):
    cx = c_ref[0]
    cy = c_ref[1]
    cz = c_ref[2]
    n = cx.shape[1]
    cc = jnp.concatenate([cx, cy, cz], axis=0)
    iota = jax.lax.broadcasted_iota(jnp.int32, (_B, n), 1)
    iota3 = jax.lax.broadcasted_iota(jnp.int32, (3 * _B, n), 1)
    d0 = ((cx - cx[:, 0:1]) ** 2 + (cy - cy[:, 0:1]) ** 2
          + (cz - cz[:, 0:1]) ** 2)
    return (cx, cy, cz, cc, iota, iota3, n), d0


def _fps_step(st, d):
    cx, cy, cz, cc, iota, iota3, n = st
    rmax = jnp.max(d, axis=1, keepdims=True)
    nxt = jnp.min(jnp.where(d == rmax, iota, n), axis=1, keepdims=True)
    nxt3 = jnp.concatenate([nxt, nxt, nxt], axis=0)
    msum = jnp.sum(jnp.where(iota3 == nxt3, cc, 0.0), axis=1,
                   keepdims=True)
    xn = msum[0:_B]
    yn = msum[_B:2 * _B]
    zn = msum[2 * _B:3 * _B]
    dn = (cx - xn) ** 2 + (cy - yn) ** 2 + (cz - zn) ** 2
    return jnp.minimum(d, dn), nxt


def _fps2_body(ma, mb, ca_ref, cb_ref, oa_ref, ob_ref):
    sta, d0a = _fps_prep(ca_ref)
    stb, d0b = _fps_prep(cb_ref)
    for b in range(_B):
        oa_ref[b, 0] = 0
        ob_ref[b, 0] = 0
    mx = max(ma, mb)

    def body(i, carry):
        da, db = carry
        da2, nxa = _fps_step(sta, da)
        db2, nxb = _fps_step(stb, db)
        if ma < mx:
            da2 = jnp.where(i < ma, da2, da)

            @pl.when(i < ma)
            def _():
                for b in range(_B):
                    oa_ref[b, i] = nxa[b, 0]
        else:
            for b in range(_B):
                oa_ref[b, i] = nxa[b, 0]
        if mb < mx:
            db2 = jnp.where(i < mb, db2, db)

            @pl.when(i < mb)
            def _():
                for b in range(_B):
                    ob_ref[b, i] = nxb[b, 0]
        else:
            for b in range(_B):
                ob_ref[b, i] = nxb[b, 0]
        return da2, db2

    jax.lax.fori_loop(1, mx, body, (d0a, d0b))


def _fps2(pos_a, ma, pos_b, mb):
    ca = jnp.transpose(pos_a, (2, 0, 1))
    cb = jnp.transpose(pos_b, (2, 0, 1))
    return pl.pallas_call(
        functools.partial(_fps2_body, ma, mb),
        out_specs=[pl.BlockSpec(memory_space=pltpu.SMEM),
                   pl.BlockSpec(memory_space=pltpu.SMEM)],
        out_shape=[jax.ShapeDtypeStruct((_B, ma), jnp.int32),
                   jax.ShapeDtypeStruct((_B, mb), jnp.int32)],
    )(ca, cb)


def _fps1_body(m, c_ref, out_ref):
    st, d0 = _fps_prep(c_ref)
    for b in range(_B):
        out_ref[b, 0] = 0

    def body(i, d):
        d2, nxt = _fps_step(st, d)
        for b in range(_B):
            out_ref[b, i] = nxt[b, 0]
        return d2

    jax.lax.fori_loop(1, m, body, d0)


def _fps(pos_bn3, m):
    coords = jnp.transpose(pos_bn3, (2, 0, 1))
    return pl.pallas_call(
        functools.partial(_fps1_body, m),
        out_specs=pl.BlockSpec(memory_space=pltpu.SMEM),
        out_shape=jax.ShapeDtypeStruct((_B, m), jnp.int32),
    )(coords)


def _gather_body(n, idx_ref, src_ref, out_ref):
    idx = idx_ref[0]
    src = src_ref[0]
    tm = idx.shape[0]
    iota = jax.lax.broadcasted_iota(jnp.int32, (tm, n), 1)
    oh = (iota == idx).astype(_BF)
    out_ref[0] = _oh_gather(oh, _split3(src))


def _gather_rows(src, idx):
    n, c = src.shape[1], src.shape[2]
    m = idx.shape[1]
    tm = min(m, 512)
    idx3 = idx.reshape(_B, m, 1)
    return pl.pallas_call(
        functools.partial(_gather_body, n),
        grid=(_B, m // tm),
        in_specs=[
            pl.BlockSpec((1, tm, 1), lambda b, t: (b, t, 0)),
            pl.BlockSpec((1, n, c), lambda b, t: (b, 0, 0)),
        ],
        out_specs=pl.BlockSpec((1, tm, c), lambda b, t: (b, t, 0)),
        out_shape=jax.ShapeDtypeStruct((_B, m, c), _F32),
    )(idx3, src)


def _knn_body(k, n, srcc_ref, dst_ref, idx_ref, val_ref):
    sc = srcc_ref[0]
    sx = sc[0:1, :]
    sy = sc[1:2, :]
    sz = sc[2:3, :]
    pd = dst_ref[0]
    dx = pd[:, 0:1]
    dy = pd[:, 1:2]
    dz = pd[:, 2:3]
    d2 = (dx - sx) ** 2 + (dy - sy) ** 2 + (dz - sz) ** 2
    tm = d2.shape[0]
    iota = jax.lax.broadcasted_iota(jnp.int32, (tm, n), 1)
    idx_cols = []
    val_cols = []
    for _ in range(k):
        v = jnp.min(d2, axis=1, keepdims=True)
        ix = jnp.min(jnp.where(d2 == v, iota, n), axis=1, keepdims=True)
        idx_cols.append(ix)
        val_cols.append(v)
        d2 = jnp.where(iota == ix, jnp.inf, d2)
    idx_ref[0] = jnp.concatenate(idx_cols, axis=1)
    val_ref[0] = jnp.concatenate(val_cols, axis=1)


def _knn(pos_src, pos_dst, k):
    n = pos_src.shape[1]
    m = pos_dst.shape[1]
    tm = min(m, 256)
    srcc = jnp.transpose(pos_src, (0, 2, 1))
    return pl.pallas_call(
        functools.partial(_knn_body, k, n),
        grid=(_B, m // tm),
        in_specs=[
            pl.BlockSpec((1, 3, n), lambda b, t: (b, 0, 0)),
            pl.BlockSpec((1, tm, 3), lambda b, t: (b, t, 0)),
        ],
        out_specs=[
            pl.BlockSpec((1, tm, k), lambda b, t: (b, t, 0)),
            pl.BlockSpec((1, tm, k), lambda b, t: (b, t, 0)),
        ],
        out_shape=[
            jax.ShapeDtypeStruct((_B, m, k), jnp.int32),
            jax.ShapeDtypeStruct((_B, m, k), _F32),
        ],
    )(srcc, pos_dst)


def _sa_body(k, n, has_x, nbr_ref, pd_ref, ps_ref, *rest):
    if has_x:
        (xs_ref, w1_ref, b1_ref, w2_ref, b2_ref, wsc_ref, bsc_ref,
         out_ref) = rest
        src = jnp.concatenate([xs_ref[0], ps_ref[0]], axis=1)
    else:
        w1_ref, b1_ref, w2_ref, b2_ref, wsc_ref, bsc_ref, out_ref = rest
        src = ps_ref[0]
    nbr = nbr_ref[0]
    pd = pd_ref[0]
    tm = nbr.shape[0]
    c = src.shape[1] - 3
    if c:
        pdpad = jnp.concatenate([jnp.zeros((tm, c), _F32), pd], axis=1)
    else:
        pdpad = pd
    parts = _split3(src)
    grp = min(k, 8)
    iota = jax.lax.broadcasted_iota(jnp.int32, (grp * tm, n), 1)
    pdpad_g = jnp.concatenate([pdpad] * grp, axis=0)
    w1 = w1_ref[:]
    b1 = b1_ref[:]
    w2 = w2_ref[:]
    b2 = b2_ref[:]
    wsc = wsc_ref[:]
    bsc = bsc_ref[:]
    acc = None
    for q in range(k // grp):
        idxcol = jnp.concatenate(
            [nbr[:, j:j + 1] for j in range(q * grp, (q + 1) * grp)], axis=0)
        oh = (iota == idxcol).astype(_BF)
        g = _oh_gather(oh, parts)
        h0 = g - pdpad_g
        h = _dot(_relu(_dot(h0, w1) + b1), w2) + b2
        h = h + _dot(h0, wsc) + bsc
        h = _relu(h)
        for j in range(grp):
            hj = h[j * tm:(j + 1) * tm, :]
            acc = hj if acc is None else jnp.maximum(acc, hj)
    out_ref[0] = acc


def _sa(nbr, pos_dst, pos_src, x_src, w):
    n = pos_src.shape[1]
    m = nbr.shape[1]
    k = nbr.shape[2]
    tm = min(m, 128)
    cout = w['l2']['w'].shape[1]
    has_x = x_src is not None
    cin = w['l1']['w'].shape[0]
    hid = w['l1']['w'].shape[1]
    cx = x_src.shape[2] if has_x else 0
    wspec = lambda s: pl.BlockSpec(s, lambda b, t: tuple(0 for _ in s))
    in_specs = [
        pl.BlockSpec((1, tm, k), lambda b, t: (b, t, 0)),
        pl.BlockSpec((1, tm, 3), lambda b, t: (b, t, 0)),
        pl.BlockSpec((1, n, 3), lambda b, t: (b, 0, 0)),
    ]
    args = [nbr, pos_dst, pos_src]
    if has_x:
        in_specs.append(pl.BlockSpec((1, n, cx), lambda b, t: (b, 0, 0)))
        args.append(x_src)
    in_specs += [wspec((cin, hid)), wspec((1, hid)), wspec((hid, cout)),
                 wspec((1, cout)), wspec((cin, cout)), wspec((1, cout))]
    args += [w['l1']['w'], w['l1']['b'].reshape(1, -1),
             w['l2']['w'], w['l2']['b'].reshape(1, -1),
             w['sc']['w'], w['sc']['b'].reshape(1, -1)]
    return pl.pallas_call(
        functools.partial(_sa_body, k, n, has_x),
        grid=(_B, m // tm),
        in_specs=in_specs,
        out_specs=pl.BlockSpec((1, tm, cout), lambda b, t: (b, t, 0)),
        out_shape=jax.ShapeDtypeStruct((_B, m, cout), _F32),
    )(*args)


def _fp_body(ns, has_xt, has_sc, final, srcc_ref, srcp_ref, xs_ref, *rest):
    rest = list(rest)
    xt_ref = rest.pop(0) if has_xt else None
    w1_ref, b1_ref, w2_ref, b2_ref = rest[:4]
    rest = rest[4:]
    if has_sc:
        wsc_ref, bsc_ref = rest[:2]
        rest = rest[2:]
    if final:
        f1w_ref, f1b_ref, f2w_ref, f2b_ref = rest[:4]
        rest = rest[4:]
    pt_ref = rest[0]
    out_ref = rest[1]

    sc = srcc_ref[0]
    sx = sc[0:1, :]
    sy = sc[1:2, :]
    sz = sc[2:3, :]
    pt = pt_ref[0]
    dx = pt[:, 0:1]
    dy = pt[:, 1:2]
    dz = pt[:, 2:3]
    d2 = (dx - sx) ** 2 + (dy - sy) ** 2 + (dz - sz) ** 2
    tm = d2.shape[0]
    iota = jax.lax.broadcasted_iota(jnp.int32, (tm, ns), 1)
    parts = _split3(xs_ref[0])
    num = None
    den = None
    for _ in range(3):
        v = jnp.min(d2, axis=1, keepdims=True)
        ix = jnp.min(jnp.where(d2 == v, iota, ns), axis=1, keepdims=True)
        d2 = jnp.where(iota == ix, jnp.inf, d2)
        oh = (iota == ix).astype(_BF)
        xg = _oh_gather(oh, parts)
        wgt = 1.0 / (v + 1e-16)
        contrib = xg * wgt
        num = contrib if num is None else num + contrib
        den = wgt if den is None else den + wgt
    interp = num / den
    if has_xt:
        comb = jnp.concatenate([xt_ref[0], interp], axis=1)
    else:
        comb = interp
    h = _dot(_relu(_dot(comb, w1_ref[:]) + b1_ref[:]),
             w2_ref[:]) + b2_ref[:]
    if has_sc:
        h = h + _dot(comb, wsc_ref[:]) \
            + bsc_ref[:]
    else:
        h = h + comb
    h = _relu(h)
    if final:
        h = _relu(_dot(h, f1w_ref[:])
                  + f1b_ref[:])
        h = _dot(h, f2w_ref[:]) + f2b_ref[:]
    out_ref[0] = h


def _fp(x_tgt, pos_tgt, x_src, pos_src, w, final=None):
    ns = pos_src.shape[1]
    mt = pos_tgt.shape[1]
    tm = min(mt, 256)
    c = x_src.shape[2]
    has_xt = x_tgt is not None
    has_sc = 'sc' in w
    cin = w['l1']['w'].shape[0]
    hid = w['l1']['w'].shape[1]
    cout = w['l2']['w'].shape[1]
    srcc = jnp.transpose(pos_src, (0, 2, 1))
    wspec = lambda s: pl.BlockSpec(s, lambda b, t: tuple(0 for _ in s))
    in_specs = [
        pl.BlockSpec((1, 3, ns), lambda b, t: (b, 0, 0)),
        pl.BlockSpec((1, ns, 3), lambda b, t: (b, 0, 0)),
        pl.BlockSpec((1, ns, c), lambda b, t: (b, 0, 0)),
    ]
    args = [srcc, pos_src, x_src]
    if has_xt:
        ct = x_tgt.shape[2]
        in_specs.append(pl.BlockSpec((1, tm, ct), lambda b, t: (b, t, 0)))
        args.append(x_tgt)
    in_specs += [wspec((cin, hid)), wspec((1, hid)), wspec((hid, cout)),
                 wspec((1, cout))]
    args += [w['l1']['w'], w['l1']['b'].reshape(1, -1),
             w['l2']['w'], w['l2']['b'].reshape(1, -1)]
    if has_sc:
        in_specs += [wspec((cin, cout)), wspec((1, cout))]
        args += [w['sc']['w'], w['sc']['b'].reshape(1, -1)]
    cfin = cout
    if final is not None:
        f1, f2 = final
        h1 = f1['w'].shape[1]
        cfin = f2['w'].shape[1]
        in_specs += [wspec((cout, h1)), wspec((1, h1)),
                     wspec((h1, cfin)), wspec((1, cfin))]
        args += [f1['w'], f1['b'].reshape(1, -1),
                 f2['w'], f2['b'].reshape(1, -1)]
    in_specs.append(pl.BlockSpec((1, tm, 3), lambda b, t: (b, t, 0)))
    args.append(pos_tgt)
    return pl.pallas_call(
        functools.partial(_fp_body, ns, has_xt, has_sc, final is not None),
        grid=(_B, mt // tm),
        in_specs=in_specs,
        out_specs=pl.BlockSpec((1, tm, cfin), lambda b, t: (b, t, 0)),
        out_shape=jax.ShapeDtypeStruct((_B, mt, cfin), _F32),
    )(*args)


def _head_body(x4_ref, w1_ref, b1_ref, w2_ref, b2_ref, gw_ref, gb_ref,
               out_ref):
    g = jnp.max(x4_ref[0], axis=0, keepdims=True)
    h = _dot(_relu(_dot(g, w1_ref[:]) + b1_ref[:]),
             w2_ref[:]) + b2_ref[:]
    h = _relu(h + g)
    out_ref[0] = _dot(h, gw_ref[:]) \
        + gb_ref[:]


def _head(x4, gm1, gm2):
    mper = x4.shape[1]
    d = x4.shape[2]
    cc = gm2['w'].shape[1]
    wspec = lambda s: pl.BlockSpec(s, lambda b: tuple(0 for _ in s))
    return pl.pallas_call(
        _head_body,
        grid=(_B,),
        in_specs=[
            pl.BlockSpec((1, mper, d), lambda b: (b, 0, 0)),
            wspec((d, d)), wspec((1, d)), wspec((d, d)), wspec((1, d)),
            wspec((d, cc)), wspec((1, cc)),
        ],
        out_specs=pl.BlockSpec((1, 1, cc), lambda b: (b, 0, 0)),
        out_shape=jax.ShapeDtypeStruct((_B, 1, cc), _F32),
    )(x4, gm1['l1']['w'], gm1['l1']['b'].reshape(1, -1),
      gm1['l2']['w'], gm1['l2']['b'].reshape(1, -1),
      gm2['w'], gm2['b'].reshape(1, -1))


def _bott_body(x3_ref, z_ref, c_ref, w1_ref, b1_ref, w2_ref, b2_ref,
               wsc_ref, bsc_ref, out_ref):
    x3 = x3_ref[0]
    m = x3.shape[0]
    zb = jnp.broadcast_to(z_ref[0], (m, z_ref.shape[2]))
    cb = jnp.broadcast_to(c_ref[0], (m, c_ref.shape[2]))
    comb = jnp.concatenate([x3, zb, cb], axis=1)
    h = _dot(_relu(_dot(comb, w1_ref[:]) + b1_ref[:]),
             w2_ref[:]) + b2_ref[:]
    h = h + _dot(comb, wsc_ref[:]) \
        + bsc_ref[:]
    out_ref[0] = _relu(h)


def _bott(x3, z, c, w):
    m = x3.shape[1]
    cx = x3.shape[2]
    zd = z.shape[2]
    cd = c.shape[2]
    cin = w['l1']['w'].shape[0]
    hid = w['l1']['w'].shape[1]
    cout = w['l2']['w'].shape[1]
    wspec = lambda s: pl.BlockSpec(s, lambda b: tuple(0 for _ in s))
    return pl.pallas_call(
        _bott_body,
        grid=(_B,),
        in_specs=[
            pl.BlockSpec((1, m, cx), lambda b: (b, 0, 0)),
            pl.BlockSpec((1, 1, zd), lambda b: (b, 0, 0)),
            pl.BlockSpec((1, 1, cd), lambda b: (b, 0, 0)),
            wspec((cin, hid)), wspec((1, hid)), wspec((hid, cout)),
            wspec((1, cout)), wspec((cin, cout)), wspec((1, cout)),
        ],
        out_specs=pl.BlockSpec((1, m, cout), lambda b: (b, 0, 0)),
        out_shape=jax.ShapeDtypeStruct((_B, m, cout), _F32),
    )(x3, z, c,
      w['l1']['w'], w['l1']['b'].reshape(1, -1),
      w['l2']['w'], w['l2']['b'].reshape(1, -1),
      w['sc']['w'], w['sc']['b'].reshape(1, -1))


def _sa_rest(x_src, pos, samp, w):
    pos_dst = _gather_rows(pos, samp)
    nbr, _ = _knn(pos, pos_dst, _KNN)
    xo = _sa(nbr, pos_dst, pos, x_src, w)
    return xo, pos_dst


def kernel(x, pos, batch, query_pos, query_pos_batch, params):
    cond = params['cond']
    dec = params['dec']
    pos0 = pos.reshape(_B, -1, 3)
    x0 = x.reshape(_B, -1, 3)
    q0 = query_pos.reshape(_B, -1, 3)

    se1, sd1 = _fps2(pos0, 1024, q0, 1024)
    xe1, pe1 = _sa_rest(x0, pos0, se1, cond['sa1'])
    xd1, pd1 = _sa_rest(None, q0, sd1, dec['sa1'])
    se2, sd2 = _fps2(pe1, 512, pd1, 256)
    xe2, pe2 = _sa_rest(xe1, pe1, se2, cond['sa2'])
    xd2, pd2 = _sa_rest(xd1, pd1, sd2, dec['sa2'])
    se3, sd3 = _fps2(pe2, 256, pd2, 64)
    xe3, pe3 = _sa_rest(xe2, pe2, se3, cond['sa3'])
    xd3, pd3 = _sa_rest(xd2, pd2, sd3, dec['sa3'])
    se4 = _fps(pe3, 128)
    xe4, _ = _sa_rest(xe3, pe3, se4, cond['sa4'])
    c = _head(xe4, cond['gm1'], cond['gm2'])

    z = jax.random.normal(jax.random.key(42), (_B, 64), dtype=_F32)
    bott = _bott(xd3, z.reshape(_B, 1, 64), c, dec['bott'])

    up3 = _fp(xd3, pd3, bott, pd3, dec['fp1'])
    up2 = _fp(xd2, pd2, up3, pd3, dec['fp2'])
    up1 = _fp(xd1, pd1, up2, pd2, dec['fp3'])
    out = _fp(None, q0, up1, pd1, dec['fp4'],
              final=(dec['f1'], dec['f2']))
    return out.reshape(-1, 3)


# speedup vs baseline: 10.5695x; 1.3967x over previous
import functools

import jax
import jax.numpy as jnp
from jax.experimental import pallas as pl
from jax.experimental.pallas import tpu as pltpu

_B = 4
_KNN = 16
_F32 = jnp.float32


def _dot(a, b):
    return jnp.dot(a, b, preferred_element_type=_F32)


def _relu(v):
    return jnp.maximum(v, 0.0)


_BF = jnp.bfloat16


def _split3(src):
    hi = src.astype(_BF)
    r1 = src - hi.astype(_F32)
    mid = r1.astype(_BF)
    lo = (r1 - mid.astype(_F32)).astype(_BF)
    return hi, mid, lo


def _oh_gather(oh_bf, parts):
    hi, mid, lo = parts
    d = jnp.dot(oh_bf, hi, preferred_element_type=_F32)
    d = d + jnp.dot(oh_bf, mid, preferred_element_type=_F32)
    return d + jnp.dot(oh_bf, lo, preferred_element_type=_F32)


def _fps_prep(c_ref):
    cx = c_ref[0]
    cy = c_ref[1]
    cz = c_ref[2]
    n = cx.shape[1]
    cc = jnp.concatenate([cx, cy, cz], axis=0)
    iota = jax.lax.broadcasted_iota(jnp.int32, (_B, n), 1)
    iota3 = jax.lax.broadcasted_iota(jnp.int32, (3 * _B, n), 1)
    d0 = ((cx - cx[:, 0:1]) ** 2 + (cy - cy[:, 0:1]) ** 2
          + (cz - cz[:, 0:1]) ** 2)
    return (cx, cy, cz, cc, iota, iota3, n), d0


def _fps_step(st, d):
    cx, cy, cz, cc, iota, iota3, n = st
    rmax = jnp.max(d, axis=1, keepdims=True)
    nxt = jnp.min(jnp.where(d == rmax, iota, n), axis=1, keepdims=True)
    nxt3 = jnp.concatenate([nxt, nxt, nxt], axis=0)
    msum = jnp.sum(jnp.where(iota3 == nxt3, cc, 0.0), axis=1,
                   keepdims=True)
    xn = msum[0:_B]
    yn = msum[_B:2 * _B]
    zn = msum[2 * _B:3 * _B]
    dn = (cx - xn) ** 2 + (cy - yn) ** 2 + (cz - zn) ** 2
    return jnp.minimum(d, dn), nxt


def _fps2_body(ma, mb, ca_ref, cb_ref, oa_ref, ob_ref):
    sta, d0a = _fps_prep(ca_ref)
    stb, d0b = _fps_prep(cb_ref)
    for b in range(_B):
        oa_ref[b, 0] = 0
        ob_ref[b, 0] = 0
    mx = max(ma, mb)

    def body(i, carry):
        da, db = carry
        da2, nxa = _fps_step(sta, da)
        db2, nxb = _fps_step(stb, db)
        if ma < mx:
            da2 = jnp.where(i < ma, da2, da)

            @pl.when(i < ma)
            def _():
                for b in range(_B):
                    oa_ref[b, i] = nxa[b, 0]
        else:
            for b in range(_B):
                oa_ref[b, i] = nxa[b, 0]
        if mb < mx:
            db2 = jnp.where(i < mb, db2, db)

            @pl.when(i < mb)
            def _():
                for b in range(_B):
                    ob_ref[b, i] = nxb[b, 0]
        else:
            for b in range(_B):
                ob_ref[b, i] = nxb[b, 0]
        return da2, db2

    jax.lax.fori_loop(1, mx, body, (d0a, d0b))


def _fps2(pos_a, ma, pos_b, mb):
    ca = jnp.transpose(pos_a, (2, 0, 1))
    cb = jnp.transpose(pos_b, (2, 0, 1))
    return pl.pallas_call(
        functools.partial(_fps2_body, ma, mb),
        out_specs=[pl.BlockSpec(memory_space=pltpu.SMEM),
                   pl.BlockSpec(memory_space=pltpu.SMEM)],
        out_shape=[jax.ShapeDtypeStruct((_B, ma), jnp.int32),
                   jax.ShapeDtypeStruct((_B, mb), jnp.int32)],
    )(ca, cb)


def _fps1_body(m, c_ref, out_ref):
    st, d0 = _fps_prep(c_ref)
    for b in range(_B):
        out_ref[b, 0] = 0

    def body(i, d):
        d2, nxt = _fps_step(st, d)
        for b in range(_B):
            out_ref[b, i] = nxt[b, 0]
        return d2

    jax.lax.fori_loop(1, m, body, d0)


def _fps(pos_bn3, m):
    coords = jnp.transpose(pos_bn3, (2, 0, 1))
    return pl.pallas_call(
        functools.partial(_fps1_body, m),
        out_specs=pl.BlockSpec(memory_space=pltpu.SMEM),
        out_shape=jax.ShapeDtypeStruct((_B, m), jnp.int32),
    )(coords)


def _gather_body(n, idx_ref, src_ref, out_ref):
    idx = idx_ref[0]
    src = src_ref[0]
    tm = idx.shape[0]
    iota = jax.lax.broadcasted_iota(jnp.int32, (tm, n), 1)
    oh = (iota == idx).astype(_BF)
    out_ref[0] = _oh_gather(oh, _split3(src))


def _gather_rows(src, idx):
    n, c = src.shape[1], src.shape[2]
    m = idx.shape[1]
    tm = min(m, 512)
    idx3 = idx.reshape(_B, m, 1)
    return pl.pallas_call(
        functools.partial(_gather_body, n),
        grid=(_B, m // tm),
        in_specs=[
            pl.BlockSpec((1, tm, 1), lambda b, t: (b, t, 0)),
            pl.BlockSpec((1, n, c), lambda b, t: (b, 0, 0)),
        ],
        out_specs=pl.BlockSpec((1, tm, c), lambda b, t: (b, t, 0)),
        out_shape=jax.ShapeDtypeStruct((_B, m, c), _F32),
    )(idx3, src)


def _knn_body(k, n, srcc_ref, dst_ref, idx_ref, val_ref):
    sc = srcc_ref[0]
    sx = sc[0:1, :]
    sy = sc[1:2, :]
    sz = sc[2:3, :]
    pd = dst_ref[0]
    dx = pd[:, 0:1]
    dy = pd[:, 1:2]
    dz = pd[:, 2:3]
    d2 = (dx - sx) ** 2 + (dy - sy) ** 2 + (dz - sz) ** 2
    tm = d2.shape[0]
    iota = jax.lax.broadcasted_iota(jnp.int32, (tm, n), 1)
    idx_cols = []
    val_cols = []
    for _ in range(k):
        v = jnp.min(d2, axis=1, keepdims=True)
        ix = jnp.min(jnp.where(d2 == v, iota, n), axis=1, keepdims=True)
        idx_cols.append(ix)
        val_cols.append(v)
        d2 = jnp.where(iota == ix, jnp.inf, d2)
    idx_ref[0] = jnp.concatenate(idx_cols, axis=1)
    val_ref[0] = jnp.concatenate(val_cols, axis=1)


def _knn(pos_src, pos_dst, k):
    n = pos_src.shape[1]
    m = pos_dst.shape[1]
    tm = min(m, 256)
    srcc = jnp.transpose(pos_src, (0, 2, 1))
    return pl.pallas_call(
        functools.partial(_knn_body, k, n),
        grid=(_B, m // tm),
        in_specs=[
            pl.BlockSpec((1, 3, n), lambda b, t: (b, 0, 0)),
            pl.BlockSpec((1, tm, 3), lambda b, t: (b, t, 0)),
        ],
        out_specs=[
            pl.BlockSpec((1, tm, k), lambda b, t: (b, t, 0)),
            pl.BlockSpec((1, tm, k), lambda b, t: (b, t, 0)),
        ],
        out_shape=[
            jax.ShapeDtypeStruct((_B, m, k), jnp.int32),
            jax.ShapeDtypeStruct((_B, m, k), _F32),
        ],
    )(srcc, pos_dst)


def _sa_body_small(k, n, cin, nbr_ref, pd_ref, catt_ref, w1_ref, b1_ref,
                   w2_ref, b2_ref, wsc_ref, bsc_ref, out_ref):
    catt = catt_ref[0]
    nbr = nbr_ref[0]
    pd = pd_ref[0]
    tm = nbr.shape[0]
    if cin > 3:
        pdpad = jnp.concatenate(
            [jnp.zeros((tm, cin - 3), _F32), pd], axis=1)
    else:
        pdpad = pd
    grp = min(k, 8)
    pdpad_g = jnp.concatenate([pdpad] * grp, axis=0)
    iota = jax.lax.broadcasted_iota(jnp.int32, (tm, n), 1)
    rows = [catt[c:c + 1, :] for c in range(cin)]
    w1 = w1_ref[:]
    b1 = b1_ref[:]
    w2 = w2_ref[:]
    b2 = b2_ref[:]
    wsc = wsc_ref[:]
    bsc = bsc_ref[:]
    hs = []
    for j in range(k):
        sel = iota == nbr[:, j:j + 1]
        cols = [jnp.sum(jnp.where(sel, r, 0.0), axis=1, keepdims=True)
                for r in rows]
        hs.append(jnp.concatenate(cols, axis=1))
    acc = None
    for q in range(k // grp):
        h0 = jnp.concatenate(hs[q * grp:(q + 1) * grp], axis=0) - pdpad_g
        h = _dot(_relu(_dot(h0, w1) + b1), w2) + b2
        h = h + _dot(h0, wsc) + bsc
        h = _relu(h)
        for j in range(grp):
            hj = h[j * tm:(j + 1) * tm, :]
            acc = hj if acc is None else jnp.maximum(acc, hj)
    out_ref[0] = acc


def _sa_body(k, n, has_x, nbr_ref, pd_ref, ps_ref, *rest):
    if has_x:
        (xs_ref, w1_ref, b1_ref, w2_ref, b2_ref, wsc_ref, bsc_ref,
         out_ref) = rest
        src = jnp.concatenate([xs_ref[0], ps_ref[0]], axis=1)
    else:
        w1_ref, b1_ref, w2_ref, b2_ref, wsc_ref, bsc_ref, out_ref = rest
        src = ps_ref[0]
    nbr = nbr_ref[0]
    pd = pd_ref[0]
    tm = nbr.shape[0]
    c = src.shape[1] - 3
    if c:
        pdpad = jnp.concatenate([jnp.zeros((tm, c), _F32), pd], axis=1)
    else:
        pdpad = pd
    parts = _split3(src)
    grp = min(k, 8)
    iota = jax.lax.broadcasted_iota(jnp.int32, (grp * tm, n), 1)
    pdpad_g = jnp.concatenate([pdpad] * grp, axis=0)
    w1 = w1_ref[:]
    b1 = b1_ref[:]
    w2 = w2_ref[:]
    b2 = b2_ref[:]
    wsc = wsc_ref[:]
    bsc = bsc_ref[:]
    acc = None
    for q in range(k // grp):
        idxcol = jnp.concatenate(
            [nbr[:, j:j + 1] for j in range(q * grp, (q + 1) * grp)], axis=0)
        oh = (iota == idxcol).astype(_BF)
        g = _oh_gather(oh, parts)
        h0 = g - pdpad_g
        h = _dot(_relu(_dot(h0, w1) + b1), w2) + b2
        h = h + _dot(h0, wsc) + bsc
        h = _relu(h)
        for j in range(grp):
            hj = h[j * tm:(j + 1) * tm, :]
            acc = hj if acc is None else jnp.maximum(acc, hj)
    out_ref[0] = acc


def _sa(nbr, pos_dst, pos_src, x_src, w):
    n = pos_src.shape[1]
    m = nbr.shape[1]
    k = nbr.shape[2]
    tm = min(m, 128)
    cout = w['l2']['w'].shape[1]
    has_x = x_src is not None
    cin = w['l1']['w'].shape[0]
    hid = w['l1']['w'].shape[1]
    cx = x_src.shape[2] if has_x else 0
    wspec = lambda s: pl.BlockSpec(s, lambda b, t: tuple(0 for _ in s))
    if cin <= 8:
        cat = (jnp.concatenate([x_src, pos_src], axis=2) if has_x
               else pos_src)
        catt = jnp.transpose(cat, (0, 2, 1))
        return pl.pallas_call(
            functools.partial(_sa_body_small, k, n, cin),
            grid=(_B, m // tm),
            in_specs=[
                pl.BlockSpec((1, tm, k), lambda b, t: (b, t, 0)),
                pl.BlockSpec((1, tm, 3), lambda b, t: (b, t, 0)),
                pl.BlockSpec((1, cin, n), lambda b, t: (b, 0, 0)),
                wspec((cin, hid)), wspec((1, hid)), wspec((hid, cout)),
                wspec((1, cout)), wspec((cin, cout)), wspec((1, cout)),
            ],
            out_specs=pl.BlockSpec((1, tm, cout), lambda b, t: (b, t, 0)),
            out_shape=jax.ShapeDtypeStruct((_B, m, cout), _F32),
        )(nbr, pos_dst, catt,
          w['l1']['w'], w['l1']['b'].reshape(1, -1),
          w['l2']['w'], w['l2']['b'].reshape(1, -1),
          w['sc']['w'], w['sc']['b'].reshape(1, -1))
    in_specs = [
        pl.BlockSpec((1, tm, k), lambda b, t: (b, t, 0)),
        pl.BlockSpec((1, tm, 3), lambda b, t: (b, t, 0)),
        pl.BlockSpec((1, n, 3), lambda b, t: (b, 0, 0)),
    ]
    args = [nbr, pos_dst, pos_src]
    if has_x:
        in_specs.append(pl.BlockSpec((1, n, cx), lambda b, t: (b, 0, 0)))
        args.append(x_src)
    in_specs += [wspec((cin, hid)), wspec((1, hid)), wspec((hid, cout)),
                 wspec((1, cout)), wspec((cin, cout)), wspec((1, cout))]
    args += [w['l1']['w'], w['l1']['b'].reshape(1, -1),
             w['l2']['w'], w['l2']['b'].reshape(1, -1),
             w['sc']['w'], w['sc']['b'].reshape(1, -1)]
    return pl.pallas_call(
        functools.partial(_sa_body, k, n, has_x),
        grid=(_B, m // tm),
        in_specs=in_specs,
        out_specs=pl.BlockSpec((1, tm, cout), lambda b, t: (b, t, 0)),
        out_shape=jax.ShapeDtypeStruct((_B, m, cout), _F32),
    )(*args)


def _fp_body(ns, has_xt, has_sc, final, srcc_ref, srcp_ref, xs_ref, *rest):
    rest = list(rest)
    xt_ref = rest.pop(0) if has_xt else None
    w1_ref, b1_ref, w2_ref, b2_ref = rest[:4]
    rest = rest[4:]
    if has_sc:
        wsc_ref, bsc_ref = rest[:2]
        rest = rest[2:]
    if final:
        f1w_ref, f1b_ref, f2w_ref, f2b_ref = rest[:4]
        rest = rest[4:]
    pt_ref = rest[0]
    out_ref = rest[1]

    sc = srcc_ref[0]
    sx = sc[0:1, :]
    sy = sc[1:2, :]
    sz = sc[2:3, :]
    pt = pt_ref[0]
    dx = pt[:, 0:1]
    dy = pt[:, 1:2]
    dz = pt[:, 2:3]
    d2 = (dx - sx) ** 2 + (dy - sy) ** 2 + (dz - sz) ** 2
    tm = d2.shape[0]
    iota = jax.lax.broadcasted_iota(jnp.int32, (tm, ns), 1)
    parts = _split3(xs_ref[0])
    num = None
    den = None
    for _ in range(3):
        v = jnp.min(d2, axis=1, keepdims=True)
        ix = jnp.min(jnp.where(d2 == v, iota, ns), axis=1, keepdims=True)
        d2 = jnp.where(iota == ix, jnp.inf, d2)
        oh = (iota == ix).astype(_BF)
        xg = _oh_gather(oh, parts)
        wgt = 1.0 / (v + 1e-16)
        contrib = xg * wgt
        num = contrib if num is None else num + contrib
        den = wgt if den is None else den + wgt
    interp = num / den
    if has_xt:
        comb = jnp.concatenate([xt_ref[0], interp], axis=1)
    else:
        comb = interp
    h = _dot(_relu(_dot(comb, w1_ref[:]) + b1_ref[:]),
             w2_ref[:]) + b2_ref[:]
    if has_sc:
        h = h + _dot(comb, wsc_ref[:]) \
            + bsc_ref[:]
    else:
        h = h + comb
    h = _relu(h)
    if final:
        h = _relu(_dot(h, f1w_ref[:])
                  + f1b_ref[:])
        h = _dot(h, f2w_ref[:]) + f2b_ref[:]
    out_ref[0] = h


def _fp(x_tgt, pos_tgt, x_src, pos_src, w, final=None):
    ns = pos_src.shape[1]
    mt = pos_tgt.shape[1]
    tm = min(mt, 256)
    c = x_src.shape[2]
    has_xt = x_tgt is not None
    has_sc = 'sc' in w
    cin = w['l1']['w'].shape[0]
    hid = w['l1']['w'].shape[1]
    cout = w['l2']['w'].shape[1]
    srcc = jnp.transpose(pos_src, (0, 2, 1))
    wspec = lambda s: pl.BlockSpec(s, lambda b, t: tuple(0 for _ in s))
    in_specs = [
        pl.BlockSpec((1, 3, ns), lambda b, t: (b, 0, 0)),
        pl.BlockSpec((1, ns, 3), lambda b, t: (b, 0, 0)),
        pl.BlockSpec((1, ns, c), lambda b, t: (b, 0, 0)),
    ]
    args = [srcc, pos_src, x_src]
    if has_xt:
        ct = x_tgt.shape[2]
        in_specs.append(pl.BlockSpec((1, tm, ct), lambda b, t: (b, t, 0)))
        args.append(x_tgt)
    in_specs += [wspec((cin, hid)), wspec((1, hid)), wspec((hid, cout)),
                 wspec((1, cout))]
    args += [w['l1']['w'], w['l1']['b'].reshape(1, -1),
             w['l2']['w'], w['l2']['b'].reshape(1, -1)]
    if has_sc:
        in_specs += [wspec((cin, cout)), wspec((1, cout))]
        args += [w['sc']['w'], w['sc']['b'].reshape(1, -1)]
    cfin = cout
    if final is not None:
        f1, f2 = final
        h1 = f1['w'].shape[1]
        cfin = f2['w'].shape[1]
        in_specs += [wspec((cout, h1)), wspec((1, h1)),
                     wspec((h1, cfin)), wspec((1, cfin))]
        args += [f1['w'], f1['b'].reshape(1, -1),
                 f2['w'], f2['b'].reshape(1, -1)]
    in_specs.append(pl.BlockSpec((1, tm, 3), lambda b, t: (b, t, 0)))
    args.append(pos_tgt)
    return pl.pallas_call(
        functools.partial(_fp_body, ns, has_xt, has_sc, final is not None),
        grid=(_B, mt // tm),
        in_specs=in_specs,
        out_specs=pl.BlockSpec((1, tm, cfin), lambda b, t: (b, t, 0)),
        out_shape=jax.ShapeDtypeStruct((_B, mt, cfin), _F32),
    )(*args)


def _head_body(x4_ref, w1_ref, b1_ref, w2_ref, b2_ref, gw_ref, gb_ref,
               out_ref):
    g = jnp.max(x4_ref[0], axis=0, keepdims=True)
    h = _dot(_relu(_dot(g, w1_ref[:]) + b1_ref[:]),
             w2_ref[:]) + b2_ref[:]
    h = _relu(h + g)
    out_ref[0] = _dot(h, gw_ref[:]) \
        + gb_ref[:]


def _head(x4, gm1, gm2):
    mper = x4.shape[1]
    d = x4.shape[2]
    cc = gm2['w'].shape[1]
    wspec = lambda s: pl.BlockSpec(s, lambda b: tuple(0 for _ in s))
    return pl.pallas_call(
        _head_body,
        grid=(_B,),
        in_specs=[
            pl.BlockSpec((1, mper, d), lambda b: (b, 0, 0)),
            wspec((d, d)), wspec((1, d)), wspec((d, d)), wspec((1, d)),
            wspec((d, cc)), wspec((1, cc)),
        ],
        out_specs=pl.BlockSpec((1, 1, cc), lambda b: (b, 0, 0)),
        out_shape=jax.ShapeDtypeStruct((_B, 1, cc), _F32),
    )(x4, gm1['l1']['w'], gm1['l1']['b'].reshape(1, -1),
      gm1['l2']['w'], gm1['l2']['b'].reshape(1, -1),
      gm2['w'], gm2['b'].reshape(1, -1))


def _bott_body(x3_ref, z_ref, c_ref, w1_ref, b1_ref, w2_ref, b2_ref,
               wsc_ref, bsc_ref, out_ref):
    x3 = x3_ref[0]
    m = x3.shape[0]
    zb = jnp.broadcast_to(z_ref[0], (m, z_ref.shape[2]))
    cb = jnp.broadcast_to(c_ref[0], (m, c_ref.shape[2]))
    comb = jnp.concatenate([x3, zb, cb], axis=1)
    h = _dot(_relu(_dot(comb, w1_ref[:]) + b1_ref[:]),
             w2_ref[:]) + b2_ref[:]
    h = h + _dot(comb, wsc_ref[:]) \
        + bsc_ref[:]
    out_ref[0] = _relu(h)


def _bott(x3, z, c, w):
    m = x3.shape[1]
    cx = x3.shape[2]
    zd = z.shape[2]
    cd = c.shape[2]
    cin = w['l1']['w'].shape[0]
    hid = w['l1']['w'].shape[1]
    cout = w['l2']['w'].shape[1]
    wspec = lambda s: pl.BlockSpec(s, lambda b: tuple(0 for _ in s))
    return pl.pallas_call(
        _bott_body,
        grid=(_B,),
        in_specs=[
            pl.BlockSpec((1, m, cx), lambda b: (b, 0, 0)),
            pl.BlockSpec((1, 1, zd), lambda b: (b, 0, 0)),
            pl.BlockSpec((1, 1, cd), lambda b: (b, 0, 0)),
            wspec((cin, hid)), wspec((1, hid)), wspec((hid, cout)),
            wspec((1, cout)), wspec((cin, cout)), wspec((1, cout)),
        ],
        out_specs=pl.BlockSpec((1, m, cout), lambda b: (b, 0, 0)),
        out_shape=jax.ShapeDtypeStruct((_B, m, cout), _F32),
    )(x3, z, c,
      w['l1']['w'], w['l1']['b'].reshape(1, -1),
      w['l2']['w'], w['l2']['b'].reshape(1, -1),
      w['sc']['w'], w['sc']['b'].reshape(1, -1))


def _sa_rest(x_src, pos, samp, w):
    pos_dst = _gather_rows(pos, samp)
    nbr, _ = _knn(pos, pos_dst, _KNN)
    xo = _sa(nbr, pos_dst, pos, x_src, w)
    return xo, pos_dst


def kernel(x, pos, batch, query_pos, query_pos_batch, params):
    cond = params['cond']
    dec = params['dec']
    pos0 = pos.reshape(_B, -1, 3)
    x0 = x.reshape(_B, -1, 3)
    q0 = query_pos.reshape(_B, -1, 3)

    se1, sd1 = _fps2(pos0, 1024, q0, 1024)
    xe1, pe1 = _sa_rest(x0, pos0, se1, cond['sa1'])
    xd1, pd1 = _sa_rest(None, q0, sd1, dec['sa1'])
    se2, sd2 = _fps2(pe1, 512, pd1, 256)
    xe2, pe2 = _sa_rest(xe1, pe1, se2, cond['sa2'])
    xd2, pd2 = _sa_rest(xd1, pd1, sd2, dec['sa2'])
    se3, sd3 = _fps2(pe2, 256, pd2, 64)
    xe3, pe3 = _sa_rest(xe2, pe2, se3, cond['sa3'])
    xd3, pd3 = _sa_rest(xd2, pd2, sd3, dec['sa3'])
    se4 = _fps(pe3, 128)
    xe4, _ = _sa_rest(xe3, pe3, se4, cond['sa4'])
    c = _head(xe4, cond['gm1'], cond['gm2'])

    z = jax.random.normal(jax.random.key(42), (_B, 64), dtype=_F32)
    bott = _bott(xd3, z.reshape(_B, 1, 64), c, dec['bott'])

    up3 = _fp(xd3, pd3, bott, pd3, dec['fp1'])
    up2 = _fp(xd2, pd2, up3, pd3, dec['fp2'])
    up1 = _fp(xd1, pd1, up2, pd2, dec['fp3'])
    out = _fp(None, q0, up1, pd1, dec['fp4'],
              final=(dec['f1'], dec['f2']))
    return out.reshape(-1, 3)


# speedup vs baseline: 13.5095x; 1.2782x over previous
import functools

import jax
import jax.numpy as jnp
from jax import lax
from jax.experimental import pallas as pl
from jax.experimental.pallas import tpu as pltpu
from jax.experimental.pallas import tpu_sc as plsc

_B = 4
_KNN = 16
_F32 = jnp.float32


def _dot(a, b):
    return jnp.dot(a, b, preferred_element_type=_F32)


def _relu(v):
    return jnp.maximum(v, 0.0)


_BF = jnp.bfloat16


def _split3(src):
    hi = src.astype(_BF)
    r1 = src - hi.astype(_F32)
    mid = r1.astype(_BF)
    lo = (r1 - mid.astype(_F32)).astype(_BF)
    return hi, mid, lo


def _oh_gather(oh_bf, parts):
    hi, mid, lo = parts
    d = jnp.dot(oh_bf, hi, preferred_element_type=_F32)
    d = d + jnp.dot(oh_bf, mid, preferred_element_type=_F32)
    return d + jnp.dot(oh_bf, lo, preferred_element_type=_F32)


def _fps_prep(c_ref):
    cx = c_ref[0]
    cy = c_ref[1]
    cz = c_ref[2]
    n = cx.shape[1]
    cc = jnp.concatenate([cx, cy, cz], axis=0)
    iota = jax.lax.broadcasted_iota(jnp.int32, (_B, n), 1)
    iota3 = jax.lax.broadcasted_iota(jnp.int32, (3 * _B, n), 1)
    d0 = ((cx - cx[:, 0:1]) ** 2 + (cy - cy[:, 0:1]) ** 2
          + (cz - cz[:, 0:1]) ** 2)
    return (cx, cy, cz, cc, iota, iota3, n), d0


def _fps_step(st, d):
    cx, cy, cz, cc, iota, iota3, n = st
    rmax = jnp.max(d, axis=1, keepdims=True)
    nxt = jnp.min(jnp.where(d == rmax, iota, n), axis=1, keepdims=True)
    nxt3 = jnp.concatenate([nxt, nxt, nxt], axis=0)
    msum = jnp.sum(jnp.where(iota3 == nxt3, cc, 0.0), axis=1,
                   keepdims=True)
    xn = msum[0:_B]
    yn = msum[_B:2 * _B]
    zn = msum[2 * _B:3 * _B]
    dn = (cx - xn) ** 2 + (cy - yn) ** 2 + (cz - zn) ** 2
    return jnp.minimum(d, dn), nxt


def _fps2_body(ma, mb, ca_ref, cb_ref, oa_ref, ob_ref):
    sta, d0a = _fps_prep(ca_ref)
    stb, d0b = _fps_prep(cb_ref)
    for b in range(_B):
        oa_ref[b, 0] = 0
        ob_ref[b, 0] = 0
    mx = max(ma, mb)

    def body(i, carry):
        da, db = carry
        da2, nxa = _fps_step(sta, da)
        db2, nxb = _fps_step(stb, db)
        if ma < mx:
            da2 = jnp.where(i < ma, da2, da)

            @pl.when(i < ma)
            def _():
                for b in range(_B):
                    oa_ref[b, i] = nxa[b, 0]
        else:
            for b in range(_B):
                oa_ref[b, i] = nxa[b, 0]
        if mb < mx:
            db2 = jnp.where(i < mb, db2, db)

            @pl.when(i < mb)
            def _():
                for b in range(_B):
                    ob_ref[b, i] = nxb[b, 0]
        else:
            for b in range(_B):
                ob_ref[b, i] = nxb[b, 0]
        return da2, db2

    jax.lax.fori_loop(1, mx, body, (d0a, d0b))


def _fps2(pos_a, ma, pos_b, mb):
    ca = jnp.transpose(pos_a, (2, 0, 1))
    cb = jnp.transpose(pos_b, (2, 0, 1))
    return pl.pallas_call(
        functools.partial(_fps2_body, ma, mb),
        out_specs=[pl.BlockSpec(memory_space=pltpu.SMEM),
                   pl.BlockSpec(memory_space=pltpu.SMEM)],
        out_shape=[jax.ShapeDtypeStruct((_B, ma), jnp.int32),
                   jax.ShapeDtypeStruct((_B, mb), jnp.int32)],
    )(ca, cb)


def _fps1_body(m, c_ref, out_ref):
    st, d0 = _fps_prep(c_ref)
    for b in range(_B):
        out_ref[b, 0] = 0

    def body(i, d):
        d2, nxt = _fps_step(st, d)
        for b in range(_B):
            out_ref[b, i] = nxt[b, 0]
        return d2

    jax.lax.fori_loop(1, m, body, d0)


def _fps(pos_bn3, m):
    coords = jnp.transpose(pos_bn3, (2, 0, 1))
    return pl.pallas_call(
        functools.partial(_fps1_body, m),
        out_specs=pl.BlockSpec(memory_space=pltpu.SMEM),
        out_shape=jax.ShapeDtypeStruct((_B, m), jnp.int32),
    )(coords)


def _gather_body(n, idx_ref, src_ref, out_ref):
    idx = idx_ref[0]
    src = src_ref[0]
    tm = idx.shape[0]
    iota = jax.lax.broadcasted_iota(jnp.int32, (tm, n), 1)
    oh = (iota == idx).astype(_BF)
    out_ref[0] = _oh_gather(oh, _split3(src))


def _gather_rows(src, idx):
    n, c = src.shape[1], src.shape[2]
    m = idx.shape[1]
    tm = min(m, 512)
    idx3 = idx.reshape(_B, m, 1)
    return pl.pallas_call(
        functools.partial(_gather_body, n),
        grid=(_B, m // tm),
        in_specs=[
            pl.BlockSpec((1, tm, 1), lambda b, t: (b, t, 0)),
            pl.BlockSpec((1, n, c), lambda b, t: (b, 0, 0)),
        ],
        out_specs=pl.BlockSpec((1, tm, c), lambda b, t: (b, t, 0)),
        out_shape=jax.ShapeDtypeStruct((_B, m, c), _F32),
    )(idx3, src)


def _knn_body(k, n, srcc_ref, dst_ref, idx_ref, val_ref):
    sc = srcc_ref[0]
    sx = sc[0:1, :]
    sy = sc[1:2, :]
    sz = sc[2:3, :]
    pd = dst_ref[0]
    dx = pd[:, 0:1]
    dy = pd[:, 1:2]
    dz = pd[:, 2:3]
    d2 = (dx - sx) ** 2 + (dy - sy) ** 2 + (dz - sz) ** 2
    tm = d2.shape[0]
    iota = jax.lax.broadcasted_iota(jnp.int32, (tm, n), 1)
    idx_cols = []
    val_cols = []
    for _ in range(k):
        v = jnp.min(d2, axis=1, keepdims=True)
        ix = jnp.min(jnp.where(d2 == v, iota, n), axis=1, keepdims=True)
        idx_cols.append(ix)
        val_cols.append(v)
        d2 = jnp.where(iota == ix, jnp.inf, d2)
    idx_ref[0] = jnp.concatenate(idx_cols, axis=1)
    val_ref[0] = jnp.concatenate(val_cols, axis=1)


def _knn(pos_src, pos_dst, k):
    n = pos_src.shape[1]
    m = pos_dst.shape[1]
    tm = min(m, 256)
    srcc = jnp.transpose(pos_src, (0, 2, 1))
    return pl.pallas_call(
        functools.partial(_knn_body, k, n),
        grid=(_B, m // tm),
        in_specs=[
            pl.BlockSpec((1, 3, n), lambda b, t: (b, 0, 0)),
            pl.BlockSpec((1, tm, 3), lambda b, t: (b, t, 0)),
        ],
        out_specs=[
            pl.BlockSpec((1, tm, k), lambda b, t: (b, t, 0)),
            pl.BlockSpec((1, tm, k), lambda b, t: (b, t, 0)),
        ],
        out_shape=[
            jax.ShapeDtypeStruct((_B, m, k), jnp.int32),
            jax.ShapeDtypeStruct((_B, m, k), _F32),
        ],
    )(srcc, pos_dst)


@functools.lru_cache(maxsize=None)
def _sc_gather_fn(v, d, e):
    info = plsc.get_sparse_core_info()
    nw = info.num_cores * info.num_subcores
    e_per_w = e // nw
    ch = min(e_per_w, 256)
    mesh = plsc.VectorSubcoreMesh(core_axis_name="c", subcore_axis_name="s")

    @functools.partial(
        pl.kernel, mesh=mesh,
        out_type=jax.ShapeDtypeStruct((e, d), jnp.float32),
        scratch_types=[
            pltpu.VMEM((ch,), jnp.int32),
            pltpu.VMEM((ch, d), jnp.float32),
            pltpu.SemaphoreType.DMA,
        ],
    )
    def k(table_hbm, idx_hbm, out_hbm, idx_v, rows_v, sem):
        wid = lax.axis_index("s") * info.num_cores + lax.axis_index("c")
        base = wid * e_per_w
        for c in range(e_per_w // ch):
            off = base + c * ch
            pltpu.sync_copy(idx_hbm.at[pl.ds(off, ch)], idx_v)
            pltpu.async_copy(table_hbm.at[idx_v], rows_v, sem).wait()
            pltpu.sync_copy(rows_v, out_hbm.at[pl.ds(off, ch)])

    return k


def _sc_gather(table, idx):
    return _sc_gather_fn(table.shape[0], table.shape[1], idx.shape[0])(
        table, idx)


def _sa_body_pre(k, cin, rows_ref, pd_ref, w1_ref, b1_ref, w2_ref, b2_ref,
                 wsc_ref, bsc_ref, out_ref):
    pd = pd_ref[0]
    tm = pd.shape[0]
    if cin > 3:
        pdpad = jnp.concatenate(
            [jnp.zeros((tm, cin - 3), _F32), pd], axis=1)
    else:
        pdpad = pd
    grp = min(k, 8)
    pdpad_g = jnp.concatenate([pdpad] * grp, axis=0)
    w1 = w1_ref[:]
    b1 = b1_ref[:]
    w2 = w2_ref[:]
    b2 = b2_ref[:]
    wsc = wsc_ref[:]
    bsc = bsc_ref[:]
    acc = None
    for q in range(k // grp):
        h0 = jnp.concatenate(
            [rows_ref[0][j][:, 0:cin]
             for j in range(q * grp, (q + 1) * grp)], axis=0) - pdpad_g
        h = _dot(_relu(_dot(h0, w1) + b1), w2) + b2
        h = h + _dot(h0, wsc) + bsc
        h = _relu(h)
        for j in range(grp):
            hj = h[j * tm:(j + 1) * tm, :]
            acc = hj if acc is None else jnp.maximum(acc, hj)
    out_ref[0] = acc
def _sa_body_small(k, n, cin, nbr_ref, pd_ref, catt_ref, w1_ref, b1_ref,
                   w2_ref, b2_ref, wsc_ref, bsc_ref, out_ref):
    catt = catt_ref[0]
    nbr = nbr_ref[0]
    pd = pd_ref[0]
    tm = nbr.shape[0]
    if cin > 3:
        pdpad = jnp.concatenate(
            [jnp.zeros((tm, cin - 3), _F32), pd], axis=1)
    else:
        pdpad = pd
    grp = min(k, 8)
    pdpad_g = jnp.concatenate([pdpad] * grp, axis=0)
    iota = jax.lax.broadcasted_iota(jnp.int32, (tm, n), 1)
    rows = [catt[c:c + 1, :] for c in range(cin)]
    w1 = w1_ref[:]
    b1 = b1_ref[:]
    w2 = w2_ref[:]
    b2 = b2_ref[:]
    wsc = wsc_ref[:]
    bsc = bsc_ref[:]
    hs = []
    for j in range(k):
        sel = iota == nbr[:, j:j + 1]
        cols = [jnp.sum(jnp.where(sel, r, 0.0), axis=1, keepdims=True)
                for r in rows]
        hs.append(jnp.concatenate(cols, axis=1))
    acc = None
    for q in range(k // grp):
        h0 = jnp.concatenate(hs[q * grp:(q + 1) * grp], axis=0) - pdpad_g
        h = _dot(_relu(_dot(h0, w1) + b1), w2) + b2
        h = h + _dot(h0, wsc) + bsc
        h = _relu(h)
        for j in range(grp):
            hj = h[j * tm:(j + 1) * tm, :]
            acc = hj if acc is None else jnp.maximum(acc, hj)
    out_ref[0] = acc


def _sa_body(k, n, has_x, nbr_ref, pd_ref, ps_ref, *rest):
    if has_x:
        (xs_ref, w1_ref, b1_ref, w2_ref, b2_ref, wsc_ref, bsc_ref,
         out_ref) = rest
        src = jnp.concatenate([xs_ref[0], ps_ref[0]], axis=1)
    else:
        w1_ref, b1_ref, w2_ref, b2_ref, wsc_ref, bsc_ref, out_ref = rest
        src = ps_ref[0]
    nbr = nbr_ref[0]
    pd = pd_ref[0]
    tm = nbr.shape[0]
    c = src.shape[1] - 3
    if c:
        pdpad = jnp.concatenate([jnp.zeros((tm, c), _F32), pd], axis=1)
    else:
        pdpad = pd
    parts = _split3(src)
    grp = min(k, 8)
    iota = jax.lax.broadcasted_iota(jnp.int32, (grp * tm, n), 1)
    pdpad_g = jnp.concatenate([pdpad] * grp, axis=0)
    w1 = w1_ref[:]
    b1 = b1_ref[:]
    w2 = w2_ref[:]
    b2 = b2_ref[:]
    wsc = wsc_ref[:]
    bsc = bsc_ref[:]
    acc = None
    for q in range(k // grp):
        idxcol = jnp.concatenate(
            [nbr[:, j:j + 1] for j in range(q * grp, (q + 1) * grp)], axis=0)
        oh = (iota == idxcol).astype(_BF)
        g = _oh_gather(oh, parts)
        h0 = g - pdpad_g
        h = _dot(_relu(_dot(h0, w1) + b1), w2) + b2
        h = h + _dot(h0, wsc) + bsc
        h = _relu(h)
        for j in range(grp):
            hj = h[j * tm:(j + 1) * tm, :]
            acc = hj if acc is None else jnp.maximum(acc, hj)
    out_ref[0] = acc


def _sa(nbr, pos_dst, pos_src, x_src, w):
    n = pos_src.shape[1]
    m = nbr.shape[1]
    k = nbr.shape[2]
    tm = min(m, 128)
    cout = w['l2']['w'].shape[1]
    has_x = x_src is not None
    cin = w['l1']['w'].shape[0]
    hid = w['l1']['w'].shape[1]
    cx = x_src.shape[2] if has_x else 0
    wspec = lambda s: pl.BlockSpec(s, lambda b, t: tuple(0 for _ in s))
    if cin <= 8:
        cat = (jnp.concatenate([x_src, pos_src], axis=2) if has_x
               else pos_src)
        table = jnp.pad(cat.reshape(_B * n, cin),
                        ((0, 0), (0, 128 - cin)))
        offs = (jnp.arange(_B, dtype=jnp.int32) * n)[:, None, None]
        idxf = (jnp.transpose(nbr, (0, 2, 1)) + offs).reshape(-1)
        rows = _sc_gather(table, idxf).reshape(_B, k, m, 128)
        return pl.pallas_call(
            functools.partial(_sa_body_pre, k, cin),
            grid=(_B, m // tm),
            in_specs=[
                pl.BlockSpec((1, k, tm, 128), lambda b, t: (b, 0, t, 0)),
                pl.BlockSpec((1, tm, 3), lambda b, t: (b, t, 0)),
                wspec((cin, hid)), wspec((1, hid)), wspec((hid, cout)),
                wspec((1, cout)), wspec((cin, cout)), wspec((1, cout)),
            ],
            out_specs=pl.BlockSpec((1, tm, cout), lambda b, t: (b, t, 0)),
            out_shape=jax.ShapeDtypeStruct((_B, m, cout), _F32),
        )(rows, pos_dst,
          w['l1']['w'], w['l1']['b'].reshape(1, -1),
          w['l2']['w'], w['l2']['b'].reshape(1, -1),
          w['sc']['w'], w['sc']['b'].reshape(1, -1))
    in_specs = [
        pl.BlockSpec((1, tm, k), lambda b, t: (b, t, 0)),
        pl.BlockSpec((1, tm, 3), lambda b, t: (b, t, 0)),
        pl.BlockSpec((1, n, 3), lambda b, t: (b, 0, 0)),
    ]
    args = [nbr, pos_dst, pos_src]
    if has_x:
        in_specs.append(pl.BlockSpec((1, n, cx), lambda b, t: (b, 0, 0)))
        args.append(x_src)
    in_specs += [wspec((cin, hid)), wspec((1, hid)), wspec((hid, cout)),
                 wspec((1, cout)), wspec((cin, cout)), wspec((1, cout))]
    args += [w['l1']['w'], w['l1']['b'].reshape(1, -1),
             w['l2']['w'], w['l2']['b'].reshape(1, -1),
             w['sc']['w'], w['sc']['b'].reshape(1, -1)]
    return pl.pallas_call(
        functools.partial(_sa_body, k, n, has_x),
        grid=(_B, m // tm),
        in_specs=in_specs,
        out_specs=pl.BlockSpec((1, tm, cout), lambda b, t: (b, t, 0)),
        out_shape=jax.ShapeDtypeStruct((_B, m, cout), _F32),
    )(*args)


def _fp_body(ns, has_xt, has_sc, final, srcc_ref, srcp_ref, xs_ref, *rest):
    rest = list(rest)
    xt_ref = rest.pop(0) if has_xt else None
    w1_ref, b1_ref, w2_ref, b2_ref = rest[:4]
    rest = rest[4:]
    if has_sc:
        wsc_ref, bsc_ref = rest[:2]
        rest = rest[2:]
    if final:
        f1w_ref, f1b_ref, f2w_ref, f2b_ref = rest[:4]
        rest = rest[4:]
    pt_ref = rest[0]
    out_ref = rest[1]

    sc = srcc_ref[0]
    sx = sc[0:1, :]
    sy = sc[1:2, :]
    sz = sc[2:3, :]
    pt = pt_ref[0]
    dx = pt[:, 0:1]
    dy = pt[:, 1:2]
    dz = pt[:, 2:3]
    d2 = (dx - sx) ** 2 + (dy - sy) ** 2 + (dz - sz) ** 2
    tm = d2.shape[0]
    iota = jax.lax.broadcasted_iota(jnp.int32, (tm, ns), 1)
    parts = _split3(xs_ref[0])
    num = None
    den = None
    for _ in range(3):
        v = jnp.min(d2, axis=1, keepdims=True)
        ix = jnp.min(jnp.where(d2 == v, iota, ns), axis=1, keepdims=True)
        d2 = jnp.where(iota == ix, jnp.inf, d2)
        oh = (iota == ix).astype(_BF)
        xg = _oh_gather(oh, parts)
        wgt = 1.0 / (v + 1e-16)
        contrib = xg * wgt
        num = contrib if num is None else num + contrib
        den = wgt if den is None else den + wgt
    interp = num / den
    if has_xt:
        comb = jnp.concatenate([xt_ref[0], interp], axis=1)
    else:
        comb = interp
    h = _dot(_relu(_dot(comb, w1_ref[:]) + b1_ref[:]),
             w2_ref[:]) + b2_ref[:]
    if has_sc:
        h = h + _dot(comb, wsc_ref[:]) \
            + bsc_ref[:]
    else:
        h = h + comb
    h = _relu(h)
    if final:
        h = _relu(_dot(h, f1w_ref[:])
                  + f1b_ref[:])
        h = _dot(h, f2w_ref[:]) + f2b_ref[:]
    out_ref[0] = h


def _fp(x_tgt, pos_tgt, x_src, pos_src, w, final=None):
    ns = pos_src.shape[1]
    mt = pos_tgt.shape[1]
    tm = min(mt, 256)
    c = x_src.shape[2]
    has_xt = x_tgt is not None
    has_sc = 'sc' in w
    cin = w['l1']['w'].shape[0]
    hid = w['l1']['w'].shape[1]
    cout = w['l2']['w'].shape[1]
    srcc = jnp.transpose(pos_src, (0, 2, 1))
    wspec = lambda s: pl.BlockSpec(s, lambda b, t: tuple(0 for _ in s))
    in_specs = [
        pl.BlockSpec((1, 3, ns), lambda b, t: (b, 0, 0)),
        pl.BlockSpec((1, ns, 3), lambda b, t: (b, 0, 0)),
        pl.BlockSpec((1, ns, c), lambda b, t: (b, 0, 0)),
    ]
    args = [srcc, pos_src, x_src]
    if has_xt:
        ct = x_tgt.shape[2]
        in_specs.append(pl.BlockSpec((1, tm, ct), lambda b, t: (b, t, 0)))
        args.append(x_tgt)
    in_specs += [wspec((cin, hid)), wspec((1, hid)), wspec((hid, cout)),
                 wspec((1, cout))]
    args += [w['l1']['w'], w['l1']['b'].reshape(1, -1),
             w['l2']['w'], w['l2']['b'].reshape(1, -1)]
    if has_sc:
        in_specs += [wspec((cin, cout)), wspec((1, cout))]
        args += [w['sc']['w'], w['sc']['b'].reshape(1, -1)]
    cfin = cout
    if final is not None:
        f1, f2 = final
        h1 = f1['w'].shape[1]
        cfin = f2['w'].shape[1]
        in_specs += [wspec((cout, h1)), wspec((1, h1)),
                     wspec((h1, cfin)), wspec((1, cfin))]
        args += [f1['w'], f1['b'].reshape(1, -1),
                 f2['w'], f2['b'].reshape(1, -1)]
    in_specs.append(pl.BlockSpec((1, tm, 3), lambda b, t: (b, t, 0)))
    args.append(pos_tgt)
    return pl.pallas_call(
        functools.partial(_fp_body, ns, has_xt, has_sc, final is not None),
        grid=(_B, mt // tm),
        in_specs=in_specs,
        out_specs=pl.BlockSpec((1, tm, cfin), lambda b, t: (b, t, 0)),
        out_shape=jax.ShapeDtypeStruct((_B, mt, cfin), _F32),
    )(*args)


def _head_body(x4_ref, w1_ref, b1_ref, w2_ref, b2_ref, gw_ref, gb_ref,
               out_ref):
    g = jnp.max(x4_ref[0], axis=0, keepdims=True)
    h = _dot(_relu(_dot(g, w1_ref[:]) + b1_ref[:]),
             w2_ref[:]) + b2_ref[:]
    h = _relu(h + g)
    out_ref[0] = _dot(h, gw_ref[:]) \
        + gb_ref[:]


def _head(x4, gm1, gm2):
    mper = x4.shape[1]
    d = x4.shape[2]
    cc = gm2['w'].shape[1]
    wspec = lambda s: pl.BlockSpec(s, lambda b: tuple(0 for _ in s))
    return pl.pallas_call(
        _head_body,
        grid=(_B,),
        in_specs=[
            pl.BlockSpec((1, mper, d), lambda b: (b, 0, 0)),
            wspec((d, d)), wspec((1, d)), wspec((d, d)), wspec((1, d)),
            wspec((d, cc)), wspec((1, cc)),
        ],
        out_specs=pl.BlockSpec((1, 1, cc), lambda b: (b, 0, 0)),
        out_shape=jax.ShapeDtypeStruct((_B, 1, cc), _F32),
    )(x4, gm1['l1']['w'], gm1['l1']['b'].reshape(1, -1),
      gm1['l2']['w'], gm1['l2']['b'].reshape(1, -1),
      gm2['w'], gm2['b'].reshape(1, -1))


def _bott_body(x3_ref, z_ref, c_ref, w1_ref, b1_ref, w2_ref, b2_ref,
               wsc_ref, bsc_ref, out_ref):
    x3 = x3_ref[0]
    m = x3.shape[0]
    zb = jnp.broadcast_to(z_ref[0], (m, z_ref.shape[2]))
    cb = jnp.broadcast_to(c_ref[0], (m, c_ref.shape[2]))
    comb = jnp.concatenate([x3, zb, cb], axis=1)
    h = _dot(_relu(_dot(comb, w1_ref[:]) + b1_ref[:]),
             w2_ref[:]) + b2_ref[:]
    h = h + _dot(comb, wsc_ref[:]) \
        + bsc_ref[:]
    out_ref[0] = _relu(h)


def _bott(x3, z, c, w):
    m = x3.shape[1]
    cx = x3.shape[2]
    zd = z.shape[2]
    cd = c.shape[2]
    cin = w['l1']['w'].shape[0]
    hid = w['l1']['w'].shape[1]
    cout = w['l2']['w'].shape[1]
    wspec = lambda s: pl.BlockSpec(s, lambda b: tuple(0 for _ in s))
    return pl.pallas_call(
        _bott_body,
        grid=(_B,),
        in_specs=[
            pl.BlockSpec((1, m, cx), lambda b: (b, 0, 0)),
            pl.BlockSpec((1, 1, zd), lambda b: (b, 0, 0)),
            pl.BlockSpec((1, 1, cd), lambda b: (b, 0, 0)),
            wspec((cin, hid)), wspec((1, hid)), wspec((hid, cout)),
            wspec((1, cout)), wspec((cin, cout)), wspec((1, cout)),
        ],
        out_specs=pl.BlockSpec((1, m, cout), lambda b: (b, 0, 0)),
        out_shape=jax.ShapeDtypeStruct((_B, m, cout), _F32),
    )(x3, z, c,
      w['l1']['w'], w['l1']['b'].reshape(1, -1),
      w['l2']['w'], w['l2']['b'].reshape(1, -1),
      w['sc']['w'], w['sc']['b'].reshape(1, -1))


def _sa_rest(x_src, pos, samp, w):
    pos_dst = _gather_rows(pos, samp)
    nbr, _ = _knn(pos, pos_dst, _KNN)
    xo = _sa(nbr, pos_dst, pos, x_src, w)
    return xo, pos_dst


def kernel(x, pos, batch, query_pos, query_pos_batch, params):
    cond = params['cond']
    dec = params['dec']
    pos0 = pos.reshape(_B, -1, 3)
    x0 = x.reshape(_B, -1, 3)
    q0 = query_pos.reshape(_B, -1, 3)

    se1, sd1 = _fps2(pos0, 1024, q0, 1024)
    xe1, pe1 = _sa_rest(x0, pos0, se1, cond['sa1'])
    xd1, pd1 = _sa_rest(None, q0, sd1, dec['sa1'])
    se2, sd2 = _fps2(pe1, 512, pd1, 256)
    xe2, pe2 = _sa_rest(xe1, pe1, se2, cond['sa2'])
    xd2, pd2 = _sa_rest(xd1, pd1, sd2, dec['sa2'])
    se3, sd3 = _fps2(pe2, 256, pd2, 64)
    xe3, pe3 = _sa_rest(xe2, pe2, se3, cond['sa3'])
    xd3, pd3 = _sa_rest(xd2, pd2, sd3, dec['sa3'])
    se4 = _fps(pe3, 128)
    xe4, _ = _sa_rest(xe3, pe3, se4, cond['sa4'])
    c = _head(xe4, cond['gm1'], cond['gm2'])

    z = jax.random.normal(jax.random.key(42), (_B, 64), dtype=_F32)
    bott = _bott(xd3, z.reshape(_B, 1, 64), c, dec['bott'])

    up3 = _fp(xd3, pd3, bott, pd3, dec['fp1'])
    up2 = _fp(xd2, pd2, up3, pd3, dec['fp2'])
    up1 = _fp(xd1, pd1, up2, pd2, dec['fp3'])
    out = _fp(None, q0, up1, pd1, dec['fp4'],
              final=(dec['f1'], dec['f2']))
    return out.reshape(-1, 3)


# speedup vs baseline: 14.5937x; 1.0803x over previous
import functools

import jax
import jax.numpy as jnp
from jax import lax
from jax.experimental import pallas as pl
from jax.experimental.pallas import tpu as pltpu
from jax.experimental.pallas import tpu_sc as plsc

_B = 4
_KNN = 16
_F32 = jnp.float32


def _dot(a, b):
    return jnp.dot(a, b, preferred_element_type=_F32)


def _relu(v):
    return jnp.maximum(v, 0.0)


_BF = jnp.bfloat16


def _split3(src):
    hi = src.astype(_BF)
    r1 = src - hi.astype(_F32)
    mid = r1.astype(_BF)
    lo = (r1 - mid.astype(_F32)).astype(_BF)
    return hi, mid, lo


def _oh_gather(oh_bf, parts):
    hi, mid, lo = parts
    d = jnp.dot(oh_bf, hi, preferred_element_type=_F32)
    d = d + jnp.dot(oh_bf, mid, preferred_element_type=_F32)
    return d + jnp.dot(oh_bf, lo, preferred_element_type=_F32)


def _fps_prep(c_ref):
    cx = c_ref[0]
    cy = c_ref[1]
    cz = c_ref[2]
    n = cx.shape[1]
    cc = jnp.concatenate([cx, cy, cz], axis=0)
    iota = jax.lax.broadcasted_iota(jnp.int32, (_B, n), 1)
    iota3 = jax.lax.broadcasted_iota(jnp.int32, (3 * _B, n), 1)
    d0 = ((cx - cx[:, 0:1]) ** 2 + (cy - cy[:, 0:1]) ** 2
          + (cz - cz[:, 0:1]) ** 2)
    return (cx, cy, cz, cc, iota, iota3, n), d0


def _fps_step(st, d):
    cx, cy, cz, cc, iota, iota3, n = st
    rmax = jnp.max(d, axis=1, keepdims=True)
    nxt = jnp.min(jnp.where(d == rmax, iota, n), axis=1, keepdims=True)
    nxt3 = jnp.concatenate([nxt, nxt, nxt], axis=0)
    msum = jnp.sum(jnp.where(iota3 == nxt3, cc, 0.0), axis=1,
                   keepdims=True)
    xn = msum[0:_B]
    yn = msum[_B:2 * _B]
    zn = msum[2 * _B:3 * _B]
    dn = (cx - xn) ** 2 + (cy - yn) ** 2 + (cz - zn) ** 2
    return jnp.minimum(d, dn), nxt


def _fps2_body(ma, mb, ca_ref, cb_ref, oa_ref, ob_ref):
    sta, d0a = _fps_prep(ca_ref)
    stb, d0b = _fps_prep(cb_ref)
    for b in range(_B):
        oa_ref[b, 0] = 0
        ob_ref[b, 0] = 0
    mx = max(ma, mb)

    def body(i, carry):
        da, db = carry
        da2, nxa = _fps_step(sta, da)
        db2, nxb = _fps_step(stb, db)
        if ma < mx:
            da2 = jnp.where(i < ma, da2, da)

            @pl.when(i < ma)
            def _():
                for b in range(_B):
                    oa_ref[b, i] = nxa[b, 0]
        else:
            for b in range(_B):
                oa_ref[b, i] = nxa[b, 0]
        if mb < mx:
            db2 = jnp.where(i < mb, db2, db)

            @pl.when(i < mb)
            def _():
                for b in range(_B):
                    ob_ref[b, i] = nxb[b, 0]
        else:
            for b in range(_B):
                ob_ref[b, i] = nxb[b, 0]
        return da2, db2

    jax.lax.fori_loop(1, mx, body, (d0a, d0b))


def _fps2(pos_a, ma, pos_b, mb):
    ca = jnp.transpose(pos_a, (2, 0, 1))
    cb = jnp.transpose(pos_b, (2, 0, 1))
    return pl.pallas_call(
        functools.partial(_fps2_body, ma, mb),
        out_specs=[pl.BlockSpec(memory_space=pltpu.SMEM),
                   pl.BlockSpec(memory_space=pltpu.SMEM)],
        out_shape=[jax.ShapeDtypeStruct((_B, ma), jnp.int32),
                   jax.ShapeDtypeStruct((_B, mb), jnp.int32)],
    )(ca, cb)


def _fps1_body(m, c_ref, out_ref):
    st, d0 = _fps_prep(c_ref)
    for b in range(_B):
        out_ref[b, 0] = 0

    def body(i, d):
        d2, nxt = _fps_step(st, d)
        for b in range(_B):
            out_ref[b, i] = nxt[b, 0]
        return d2

    jax.lax.fori_loop(1, m, body, d0)


def _fps(pos_bn3, m):
    coords = jnp.transpose(pos_bn3, (2, 0, 1))
    return pl.pallas_call(
        functools.partial(_fps1_body, m),
        out_specs=pl.BlockSpec(memory_space=pltpu.SMEM),
        out_shape=jax.ShapeDtypeStruct((_B, m), jnp.int32),
    )(coords)


def _gather_body(n, idx_ref, src_ref, out_ref):
    idx = idx_ref[0]
    src = src_ref[0]
    tm = idx.shape[0]
    iota = jax.lax.broadcasted_iota(jnp.int32, (tm, n), 1)
    oh = (iota == idx).astype(_BF)
    out_ref[0] = _oh_gather(oh, _split3(src))


def _gather_rows(src, idx):
    n, c = src.shape[1], src.shape[2]
    m = idx.shape[1]
    tm = min(m, 512)
    idx3 = idx.reshape(_B, m, 1)
    return pl.pallas_call(
        functools.partial(_gather_body, n),
        grid=(_B, m // tm),
        in_specs=[
            pl.BlockSpec((1, tm, 1), lambda b, t: (b, t, 0)),
            pl.BlockSpec((1, n, c), lambda b, t: (b, 0, 0)),
        ],
        out_specs=pl.BlockSpec((1, tm, c), lambda b, t: (b, t, 0)),
        out_shape=jax.ShapeDtypeStruct((_B, m, c), _F32),
    )(idx3, src)


def _knn_body(k, n, srcc_ref, dst_ref, idx_ref, val_ref):
    sc = srcc_ref[0]
    sx = sc[0:1, :]
    sy = sc[1:2, :]
    sz = sc[2:3, :]
    pd = dst_ref[0]
    dx = pd[:, 0:1]
    dy = pd[:, 1:2]
    dz = pd[:, 2:3]
    d2 = (dx - sx) ** 2 + (dy - sy) ** 2 + (dz - sz) ** 2
    tm = d2.shape[0]
    iota = jax.lax.broadcasted_iota(jnp.int32, (tm, n), 1)
    idx_cols = []
    val_cols = []
    for _ in range(k):
        v = jnp.min(d2, axis=1, keepdims=True)
        ix = jnp.min(jnp.where(d2 == v, iota, n), axis=1, keepdims=True)
        idx_cols.append(ix)
        val_cols.append(v)
        d2 = jnp.where(iota == ix, jnp.inf, d2)
    idx_ref[0] = jnp.concatenate(idx_cols, axis=1)
    val_ref[0] = jnp.concatenate(val_cols, axis=1)


def _knn(pos_src, pos_dst, k):
    n = pos_src.shape[1]
    m = pos_dst.shape[1]
    tm = min(m, 256)
    srcc = jnp.transpose(pos_src, (0, 2, 1))
    return pl.pallas_call(
        functools.partial(_knn_body, k, n),
        grid=(_B, m // tm),
        in_specs=[
            pl.BlockSpec((1, 3, n), lambda b, t: (b, 0, 0)),
            pl.BlockSpec((1, tm, 3), lambda b, t: (b, t, 0)),
        ],
        out_specs=[
            pl.BlockSpec((1, tm, k), lambda b, t: (b, t, 0)),
            pl.BlockSpec((1, tm, k), lambda b, t: (b, t, 0)),
        ],
        out_shape=[
            jax.ShapeDtypeStruct((_B, m, k), jnp.int32),
            jax.ShapeDtypeStruct((_B, m, k), _F32),
        ],
    )(srcc, pos_dst)


@functools.lru_cache(maxsize=None)
def _sc_gather_fn(v, d, e):
    info = plsc.get_sparse_core_info()
    nw = info.num_cores * info.num_subcores
    e_per_w = e // nw
    ch = min(e_per_w, 256)
    mesh = plsc.VectorSubcoreMesh(core_axis_name="c", subcore_axis_name="s")

    @functools.partial(
        pl.kernel, mesh=mesh,
        out_type=jax.ShapeDtypeStruct((e, d), jnp.float32),
        scratch_types=[
            pltpu.VMEM((ch,), jnp.int32),
            pltpu.VMEM((ch, d), jnp.float32),
            pltpu.SemaphoreType.DMA,
        ],
    )
    def k(table_hbm, idx_hbm, out_hbm, idx_v, rows_v, sem):
        wid = lax.axis_index("s") * info.num_cores + lax.axis_index("c")
        base = wid * e_per_w
        for c in range(e_per_w // ch):
            off = base + c * ch
            pltpu.sync_copy(idx_hbm.at[pl.ds(off, ch)], idx_v)
            pltpu.async_copy(table_hbm.at[idx_v], rows_v, sem).wait()
            pltpu.sync_copy(rows_v, out_hbm.at[pl.ds(off, ch)])

    return k


def _sc_gather(table, idx):
    return _sc_gather_fn(table.shape[0], table.shape[1], idx.shape[0])(
        table, idx)


def _sa_body_pre(k, cin, rows_ref, pd_ref, w1_ref, b1_ref, w2_ref, b2_ref,
                 wsc_ref, bsc_ref, out_ref):
    pd = pd_ref[0]
    tm = pd.shape[0]
    if cin > 3:
        pdpad = jnp.concatenate(
            [jnp.zeros((tm, cin - 3), _F32), pd], axis=1)
    else:
        pdpad = pd
    grp = min(k, 8)
    pdpad_g = jnp.concatenate([pdpad] * grp, axis=0)
    w1 = w1_ref[:]
    b1 = b1_ref[:]
    w2 = w2_ref[:]
    b2 = b2_ref[:]
    wsc = wsc_ref[:]
    bsc = bsc_ref[:]
    acc = None
    for q in range(k // grp):
        h0 = jnp.concatenate(
            [rows_ref[0][j][:, 0:cin]
             for j in range(q * grp, (q + 1) * grp)], axis=0) - pdpad_g
        h = _dot(_relu(_dot(h0, w1) + b1), w2) + b2
        h = h + _dot(h0, wsc) + bsc
        h = _relu(h)
        for j in range(grp):
            hj = h[j * tm:(j + 1) * tm, :]
            acc = hj if acc is None else jnp.maximum(acc, hj)
    out_ref[0] = acc
def _sa(nbr, pos_dst, pos_src, x_src, w):
    n = pos_src.shape[1]
    m = nbr.shape[1]
    k = nbr.shape[2]
    tm = min(m, 128)
    cout = w['l2']['w'].shape[1]
    cin = w['l1']['w'].shape[0]
    hid = w['l1']['w'].shape[1]
    wspec = lambda s: pl.BlockSpec(s, lambda b, t: tuple(0 for _ in s))
    cat = (jnp.concatenate([x_src, pos_src], axis=2)
           if x_src is not None else pos_src)
    dpad = -(-cin // 128) * 128
    table = jnp.pad(cat.reshape(_B * n, cin), ((0, 0), (0, dpad - cin)))
    offs = (jnp.arange(_B, dtype=jnp.int32) * n)[:, None, None]
    idxf = (jnp.transpose(nbr, (0, 2, 1)) + offs).reshape(-1)
    rows = _sc_gather(table, idxf).reshape(_B, k, m, dpad)
    return pl.pallas_call(
        functools.partial(_sa_body_pre, k, cin),
        grid=(_B, m // tm),
        in_specs=[
            pl.BlockSpec((1, k, tm, dpad), lambda b, t: (b, 0, t, 0)),
            pl.BlockSpec((1, tm, 3), lambda b, t: (b, t, 0)),
            wspec((cin, hid)), wspec((1, hid)), wspec((hid, cout)),
            wspec((1, cout)), wspec((cin, cout)), wspec((1, cout)),
        ],
        out_specs=pl.BlockSpec((1, tm, cout), lambda b, t: (b, t, 0)),
        out_shape=jax.ShapeDtypeStruct((_B, m, cout), _F32),
    )(rows, pos_dst,
      w['l1']['w'], w['l1']['b'].reshape(1, -1),
      w['l2']['w'], w['l2']['b'].reshape(1, -1),
      w['sc']['w'], w['sc']['b'].reshape(1, -1))


def _fp_body(ns, has_xt, has_sc, final, srcc_ref, srcp_ref, xs_ref, *rest):
    rest = list(rest)
    xt_ref = rest.pop(0) if has_xt else None
    w1_ref, b1_ref, w2_ref, b2_ref = rest[:4]
    rest = rest[4:]
    if has_sc:
        wsc_ref, bsc_ref = rest[:2]
        rest = rest[2:]
    if final:
        f1w_ref, f1b_ref, f2w_ref, f2b_ref = rest[:4]
        rest = rest[4:]
    pt_ref = rest[0]
    out_ref = rest[1]

    sc = srcc_ref[0]
    sx = sc[0:1, :]
    sy = sc[1:2, :]
    sz = sc[2:3, :]
    pt = pt_ref[0]
    dx = pt[:, 0:1]
    dy = pt[:, 1:2]
    dz = pt[:, 2:3]
    d2 = (dx - sx) ** 2 + (dy - sy) ** 2 + (dz - sz) ** 2
    tm = d2.shape[0]
    iota = jax.lax.broadcasted_iota(jnp.int32, (tm, ns), 1)
    parts = _split3(xs_ref[0])
    num = None
    den = None
    for _ in range(3):
        v = jnp.min(d2, axis=1, keepdims=True)
        ix = jnp.min(jnp.where(d2 == v, iota, ns), axis=1, keepdims=True)
        d2 = jnp.where(iota == ix, jnp.inf, d2)
        oh = (iota == ix).astype(_BF)
        xg = _oh_gather(oh, parts)
        wgt = 1.0 / (v + 1e-16)
        contrib = xg * wgt
        num = contrib if num is None else num + contrib
        den = wgt if den is None else den + wgt
    interp = num / den
    if has_xt:
        comb = jnp.concatenate([xt_ref[0], interp], axis=1)
    else:
        comb = interp
    h = _dot(_relu(_dot(comb, w1_ref[:]) + b1_ref[:]),
             w2_ref[:]) + b2_ref[:]
    if has_sc:
        h = h + _dot(comb, wsc_ref[:]) \
            + bsc_ref[:]
    else:
        h = h + comb
    h = _relu(h)
    if final:
        h = _relu(_dot(h, f1w_ref[:])
                  + f1b_ref[:])
        h = _dot(h, f2w_ref[:]) + f2b_ref[:]
    out_ref[0] = h


def _fp(x_tgt, pos_tgt, x_src, pos_src, w, final=None):
    ns = pos_src.shape[1]
    mt = pos_tgt.shape[1]
    tm = min(mt, 256)
    c = x_src.shape[2]
    has_xt = x_tgt is not None
    has_sc = 'sc' in w
    cin = w['l1']['w'].shape[0]
    hid = w['l1']['w'].shape[1]
    cout = w['l2']['w'].shape[1]
    srcc = jnp.transpose(pos_src, (0, 2, 1))
    wspec = lambda s: pl.BlockSpec(s, lambda b, t: tuple(0 for _ in s))
    in_specs = [
        pl.BlockSpec((1, 3, ns), lambda b, t: (b, 0, 0)),
        pl.BlockSpec((1, ns, 3), lambda b, t: (b, 0, 0)),
        pl.BlockSpec((1, ns, c), lambda b, t: (b, 0, 0)),
    ]
    args = [srcc, pos_src, x_src]
    if has_xt:
        ct = x_tgt.shape[2]
        in_specs.append(pl.BlockSpec((1, tm, ct), lambda b, t: (b, t, 0)))
        args.append(x_tgt)
    in_specs += [wspec((cin, hid)), wspec((1, hid)), wspec((hid, cout)),
                 wspec((1, cout))]
    args += [w['l1']['w'], w['l1']['b'].reshape(1, -1),
             w['l2']['w'], w['l2']['b'].reshape(1, -1)]
    if has_sc:
        in_specs += [wspec((cin, cout)), wspec((1, cout))]
        args += [w['sc']['w'], w['sc']['b'].reshape(1, -1)]
    cfin = cout
    if final is not None:
        f1, f2 = final
        h1 = f1['w'].shape[1]
        cfin = f2['w'].shape[1]
        in_specs += [wspec((cout, h1)), wspec((1, h1)),
                     wspec((h1, cfin)), wspec((1, cfin))]
        args += [f1['w'], f1['b'].reshape(1, -1),
                 f2['w'], f2['b'].reshape(1, -1)]
    in_specs.append(pl.BlockSpec((1, tm, 3), lambda b, t: (b, t, 0)))
    args.append(pos_tgt)
    return pl.pallas_call(
        functools.partial(_fp_body, ns, has_xt, has_sc, final is not None),
        grid=(_B, mt // tm),
        in_specs=in_specs,
        out_specs=pl.BlockSpec((1, tm, cfin), lambda b, t: (b, t, 0)),
        out_shape=jax.ShapeDtypeStruct((_B, mt, cfin), _F32),
    )(*args)


def _head_body(x4_ref, w1_ref, b1_ref, w2_ref, b2_ref, gw_ref, gb_ref,
               out_ref):
    g = jnp.max(x4_ref[0], axis=0, keepdims=True)
    h = _dot(_relu(_dot(g, w1_ref[:]) + b1_ref[:]),
             w2_ref[:]) + b2_ref[:]
    h = _relu(h + g)
    out_ref[0] = _dot(h, gw_ref[:]) \
        + gb_ref[:]


def _head(x4, gm1, gm2):
    mper = x4.shape[1]
    d = x4.shape[2]
    cc = gm2['w'].shape[1]
    wspec = lambda s: pl.BlockSpec(s, lambda b: tuple(0 for _ in s))
    return pl.pallas_call(
        _head_body,
        grid=(_B,),
        in_specs=[
            pl.BlockSpec((1, mper, d), lambda b: (b, 0, 0)),
            wspec((d, d)), wspec((1, d)), wspec((d, d)), wspec((1, d)),
            wspec((d, cc)), wspec((1, cc)),
        ],
        out_specs=pl.BlockSpec((1, 1, cc), lambda b: (b, 0, 0)),
        out_shape=jax.ShapeDtypeStruct((_B, 1, cc), _F32),
    )(x4, gm1['l1']['w'], gm1['l1']['b'].reshape(1, -1),
      gm1['l2']['w'], gm1['l2']['b'].reshape(1, -1),
      gm2['w'], gm2['b'].reshape(1, -1))


def _bott_body(x3_ref, z_ref, c_ref, w1_ref, b1_ref, w2_ref, b2_ref,
               wsc_ref, bsc_ref, out_ref):
    x3 = x3_ref[0]
    m = x3.shape[0]
    zb = jnp.broadcast_to(z_ref[0], (m, z_ref.shape[2]))
    cb = jnp.broadcast_to(c_ref[0], (m, c_ref.shape[2]))
    comb = jnp.concatenate([x3, zb, cb], axis=1)
    h = _dot(_relu(_dot(comb, w1_ref[:]) + b1_ref[:]),
             w2_ref[:]) + b2_ref[:]
    h = h + _dot(comb, wsc_ref[:]) \
        + bsc_ref[:]
    out_ref[0] = _relu(h)


def _bott(x3, z, c, w):
    m = x3.shape[1]
    cx = x3.shape[2]
    zd = z.shape[2]
    cd = c.shape[2]
    cin = w['l1']['w'].shape[0]
    hid = w['l1']['w'].shape[1]
    cout = w['l2']['w'].shape[1]
    wspec = lambda s: pl.BlockSpec(s, lambda b: tuple(0 for _ in s))
    return pl.pallas_call(
        _bott_body,
        grid=(_B,),
        in_specs=[
            pl.BlockSpec((1, m, cx), lambda b: (b, 0, 0)),
            pl.BlockSpec((1, 1, zd), lambda b: (b, 0, 0)),
            pl.BlockSpec((1, 1, cd), lambda b: (b, 0, 0)),
            wspec((cin, hid)), wspec((1, hid)), wspec((hid, cout)),
            wspec((1, cout)), wspec((cin, cout)), wspec((1, cout)),
        ],
        out_specs=pl.BlockSpec((1, m, cout), lambda b: (b, 0, 0)),
        out_shape=jax.ShapeDtypeStruct((_B, m, cout), _F32),
    )(x3, z, c,
      w['l1']['w'], w['l1']['b'].reshape(1, -1),
      w['l2']['w'], w['l2']['b'].reshape(1, -1),
      w['sc']['w'], w['sc']['b'].reshape(1, -1))


def _sa_rest(x_src, pos, samp, w):
    pos_dst = _gather_rows(pos, samp)
    nbr, _ = _knn(pos, pos_dst, _KNN)
    xo = _sa(nbr, pos_dst, pos, x_src, w)
    return xo, pos_dst


def kernel(x, pos, batch, query_pos, query_pos_batch, params):
    cond = params['cond']
    dec = params['dec']
    pos0 = pos.reshape(_B, -1, 3)
    x0 = x.reshape(_B, -1, 3)
    q0 = query_pos.reshape(_B, -1, 3)

    se1, sd1 = _fps2(pos0, 1024, q0, 1024)
    xe1, pe1 = _sa_rest(x0, pos0, se1, cond['sa1'])
    xd1, pd1 = _sa_rest(None, q0, sd1, dec['sa1'])
    se2, sd2 = _fps2(pe1, 512, pd1, 256)
    xe2, pe2 = _sa_rest(xe1, pe1, se2, cond['sa2'])
    xd2, pd2 = _sa_rest(xd1, pd1, sd2, dec['sa2'])
    se3, sd3 = _fps2(pe2, 256, pd2, 64)
    xe3, pe3 = _sa_rest(xe2, pe2, se3, cond['sa3'])
    xd3, pd3 = _sa_rest(xd2, pd2, sd3, dec['sa3'])
    se4 = _fps(pe3, 128)
    xe4, _ = _sa_rest(xe3, pe3, se4, cond['sa4'])
    c = _head(xe4, cond['gm1'], cond['gm2'])

    z = jax.random.normal(jax.random.key(42), (_B, 64), dtype=_F32)
    bott = _bott(xd3, z.reshape(_B, 1, 64), c, dec['bott'])

    up3 = _fp(xd3, pd3, bott, pd3, dec['fp1'])
    up2 = _fp(xd2, pd2, up3, pd3, dec['fp2'])
    up1 = _fp(xd1, pd1, up2, pd2, dec['fp3'])
    out = _fp(None, q0, up1, pd1, dec['fp4'],
              final=(dec['f1'], dec['f2']))
    return out.reshape(-1, 3)


# speedup vs baseline: 14.5982x; 1.0003x over previous
import functools

import jax
import jax.numpy as jnp
from jax import lax
from jax.experimental import pallas as pl
from jax.experimental.pallas import tpu as pltpu
from jax.experimental.pallas import tpu_sc as plsc

_B = 4
_KNN = 16
_F32 = jnp.float32


def _dot(a, b):
    return jnp.dot(a, b, preferred_element_type=_F32)


def _relu(v):
    return jnp.maximum(v, 0.0)


_BF = jnp.bfloat16


def _split3(src):
    hi = src.astype(_BF)
    r1 = src - hi.astype(_F32)
    mid = r1.astype(_BF)
    lo = (r1 - mid.astype(_F32)).astype(_BF)
    return hi, mid, lo


def _oh_gather(oh_bf, parts):
    hi, mid, lo = parts
    d = jnp.dot(oh_bf, hi, preferred_element_type=_F32)
    d = d + jnp.dot(oh_bf, mid, preferred_element_type=_F32)
    return d + jnp.dot(oh_bf, lo, preferred_element_type=_F32)


def _fps_prep(c_ref):
    cx = c_ref[0]
    cy = c_ref[1]
    cz = c_ref[2]
    n = cx.shape[1]
    cc = jnp.concatenate([cx, cy, cz], axis=0)
    iota = jax.lax.broadcasted_iota(jnp.int32, (_B, n), 1)
    iota3 = jax.lax.broadcasted_iota(jnp.int32, (3 * _B, n), 1)
    d0 = ((cx - cx[:, 0:1]) ** 2 + (cy - cy[:, 0:1]) ** 2
          + (cz - cz[:, 0:1]) ** 2)
    return (cx, cy, cz, cc, iota, iota3, n), d0


def _fps_step(st, d):
    cx, cy, cz, cc, iota, iota3, n = st
    rmax = jnp.max(d, axis=1, keepdims=True)
    nxt = jnp.min(jnp.where(d == rmax, iota, n), axis=1, keepdims=True)
    nxt3 = jnp.concatenate([nxt, nxt, nxt], axis=0)
    msum = jnp.sum(jnp.where(iota3 == nxt3, cc, 0.0), axis=1,
                   keepdims=True)
    xn = msum[0:_B]
    yn = msum[_B:2 * _B]
    zn = msum[2 * _B:3 * _B]
    dn = (cx - xn) ** 2 + (cy - yn) ** 2 + (cz - zn) ** 2
    return jnp.minimum(d, dn), nxt


def _fps2_body(ma, mb, ca_ref, cb_ref, oa_ref, ob_ref):
    sta, d0a = _fps_prep(ca_ref)
    stb, d0b = _fps_prep(cb_ref)
    for b in range(_B):
        oa_ref[b, 0] = 0
        ob_ref[b, 0] = 0
    mx = max(ma, mb)

    def body(i, carry):
        da, db = carry
        da2, nxa = _fps_step(sta, da)
        db2, nxb = _fps_step(stb, db)
        if ma < mx:
            da2 = jnp.where(i < ma, da2, da)

            @pl.when(i < ma)
            def _():
                for b in range(_B):
                    oa_ref[b, i] = nxa[b, 0]
        else:
            for b in range(_B):
                oa_ref[b, i] = nxa[b, 0]
        if mb < mx:
            db2 = jnp.where(i < mb, db2, db)

            @pl.when(i < mb)
            def _():
                for b in range(_B):
                    ob_ref[b, i] = nxb[b, 0]
        else:
            for b in range(_B):
                ob_ref[b, i] = nxb[b, 0]
        return da2, db2

    jax.lax.fori_loop(1, mx, body, (d0a, d0b))


def _fps2(pos_a, ma, pos_b, mb):
    ca = jnp.transpose(pos_a, (2, 0, 1))
    cb = jnp.transpose(pos_b, (2, 0, 1))
    return pl.pallas_call(
        functools.partial(_fps2_body, ma, mb),
        out_specs=[pl.BlockSpec(memory_space=pltpu.SMEM),
                   pl.BlockSpec(memory_space=pltpu.SMEM)],
        out_shape=[jax.ShapeDtypeStruct((_B, ma), jnp.int32),
                   jax.ShapeDtypeStruct((_B, mb), jnp.int32)],
    )(ca, cb)


def _fps1_body(m, c_ref, out_ref):
    st, d0 = _fps_prep(c_ref)
    for b in range(_B):
        out_ref[b, 0] = 0

    def body(i, d):
        d2, nxt = _fps_step(st, d)
        for b in range(_B):
            out_ref[b, i] = nxt[b, 0]
        return d2

    jax.lax.fori_loop(1, m, body, d0)


def _fps(pos_bn3, m):
    coords = jnp.transpose(pos_bn3, (2, 0, 1))
    return pl.pallas_call(
        functools.partial(_fps1_body, m),
        out_specs=pl.BlockSpec(memory_space=pltpu.SMEM),
        out_shape=jax.ShapeDtypeStruct((_B, m), jnp.int32),
    )(coords)


def _gather_body(n, idx_ref, src_ref, out_ref):
    idx = idx_ref[0]
    src = src_ref[0]
    tm = idx.shape[0]
    iota = jax.lax.broadcasted_iota(jnp.int32, (tm, n), 1)
    oh = (iota == idx).astype(_BF)
    out_ref[0] = _oh_gather(oh, _split3(src))


def _gather_rows(src, idx):
    n, c = src.shape[1], src.shape[2]
    m = idx.shape[1]
    tm = min(m, 512)
    idx3 = idx.reshape(_B, m, 1)
    return pl.pallas_call(
        functools.partial(_gather_body, n),
        grid=(_B, m // tm),
        compiler_params=pltpu.CompilerParams(
            dimension_semantics=("parallel", "arbitrary")),
        in_specs=[
            pl.BlockSpec((1, tm, 1), lambda b, t: (b, t, 0)),
            pl.BlockSpec((1, n, c), lambda b, t: (b, 0, 0)),
        ],
        out_specs=pl.BlockSpec((1, tm, c), lambda b, t: (b, t, 0)),
        out_shape=jax.ShapeDtypeStruct((_B, m, c), _F32),
    )(idx3, src)


def _knn_body(k, n, srcc_ref, dst_ref, idx_ref, val_ref):
    sc = srcc_ref[0]
    sx = sc[0:1, :]
    sy = sc[1:2, :]
    sz = sc[2:3, :]
    pd = dst_ref[0]
    dx = pd[:, 0:1]
    dy = pd[:, 1:2]
    dz = pd[:, 2:3]
    d2 = (dx - sx) ** 2 + (dy - sy) ** 2 + (dz - sz) ** 2
    tm = d2.shape[0]
    iota = jax.lax.broadcasted_iota(jnp.int32, (tm, n), 1)
    idx_cols = []
    val_cols = []
    for _ in range(k):
        v = jnp.min(d2, axis=1, keepdims=True)
        ix = jnp.min(jnp.where(d2 == v, iota, n), axis=1, keepdims=True)
        idx_cols.append(ix)
        val_cols.append(v)
        d2 = jnp.where(iota == ix, jnp.inf, d2)
    idx_ref[0] = jnp.concatenate(idx_cols, axis=1)
    val_ref[0] = jnp.concatenate(val_cols, axis=1)


def _knn(pos_src, pos_dst, k):
    n = pos_src.shape[1]
    m = pos_dst.shape[1]
    tm = min(m, 256)
    srcc = jnp.transpose(pos_src, (0, 2, 1))
    return pl.pallas_call(
        functools.partial(_knn_body, k, n),
        grid=(_B, m // tm),
        compiler_params=pltpu.CompilerParams(
            dimension_semantics=("parallel", "arbitrary")),
        in_specs=[
            pl.BlockSpec((1, 3, n), lambda b, t: (b, 0, 0)),
            pl.BlockSpec((1, tm, 3), lambda b, t: (b, t, 0)),
        ],
        out_specs=[
            pl.BlockSpec((1, tm, k), lambda b, t: (b, t, 0)),
            pl.BlockSpec((1, tm, k), lambda b, t: (b, t, 0)),
        ],
        out_shape=[
            jax.ShapeDtypeStruct((_B, m, k), jnp.int32),
            jax.ShapeDtypeStruct((_B, m, k), _F32),
        ],
    )(srcc, pos_dst)


@functools.lru_cache(maxsize=None)
def _sc_gather_fn(v, d, e):
    info = plsc.get_sparse_core_info()
    nw = info.num_cores * info.num_subcores
    e_per_w = e // nw
    ch = min(e_per_w, 256)
    mesh = plsc.VectorSubcoreMesh(core_axis_name="c", subcore_axis_name="s")

    @functools.partial(
        pl.kernel, mesh=mesh,
        out_type=jax.ShapeDtypeStruct((e, d), jnp.float32),
        scratch_types=[
            pltpu.VMEM((ch,), jnp.int32),
            pltpu.VMEM((ch, d), jnp.float32),
            pltpu.SemaphoreType.DMA,
        ],
    )
    def k(table_hbm, idx_hbm, out_hbm, idx_v, rows_v, sem):
        wid = lax.axis_index("s") * info.num_cores + lax.axis_index("c")
        base = wid * e_per_w
        for c in range(e_per_w // ch):
            off = base + c * ch
            pltpu.sync_copy(idx_hbm.at[pl.ds(off, ch)], idx_v)
            pltpu.async_copy(table_hbm.at[idx_v], rows_v, sem).wait()
            pltpu.sync_copy(rows_v, out_hbm.at[pl.ds(off, ch)])

    return k


def _sc_gather(table, idx):
    return _sc_gather_fn(table.shape[0], table.shape[1], idx.shape[0])(
        table, idx)


def _sa_body_pre(k, cin, rows_ref, pd_ref, w1_ref, b1_ref, w2_ref, b2_ref,
                 wsc_ref, bsc_ref, out_ref):
    pd = pd_ref[0]
    tm = pd.shape[0]
    if cin > 3:
        pdpad = jnp.concatenate(
            [jnp.zeros((tm, cin - 3), _F32), pd], axis=1)
    else:
        pdpad = pd
    grp = min(k, 8)
    pdpad_g = jnp.concatenate([pdpad] * grp, axis=0)
    w1 = w1_ref[:]
    b1 = b1_ref[:]
    w2 = w2_ref[:]
    b2 = b2_ref[:]
    wsc = wsc_ref[:]
    bsc = bsc_ref[:]
    acc = None
    for q in range(k // grp):
        h0 = jnp.concatenate(
            [rows_ref[0][j][:, 0:cin]
             for j in range(q * grp, (q + 1) * grp)], axis=0) - pdpad_g
        h = _dot(_relu(_dot(h0, w1) + b1), w2) + b2
        h = h + _dot(h0, wsc) + bsc
        h = _relu(h)
        for j in range(grp):
            hj = h[j * tm:(j + 1) * tm, :]
            acc = hj if acc is None else jnp.maximum(acc, hj)
    out_ref[0] = acc
def _sa(nbr, pos_dst, pos_src, x_src, w):
    n = pos_src.shape[1]
    m = nbr.shape[1]
    k = nbr.shape[2]
    tm = min(m, 128)
    cout = w['l2']['w'].shape[1]
    cin = w['l1']['w'].shape[0]
    hid = w['l1']['w'].shape[1]
    wspec = lambda s: pl.BlockSpec(s, lambda b, t: tuple(0 for _ in s))
    cat = (jnp.concatenate([x_src, pos_src], axis=2)
           if x_src is not None else pos_src)
    dpad = -(-cin // 128) * 128
    table = jnp.pad(cat.reshape(_B * n, cin), ((0, 0), (0, dpad - cin)))
    offs = (jnp.arange(_B, dtype=jnp.int32) * n)[:, None, None]
    idxf = (jnp.transpose(nbr, (0, 2, 1)) + offs).reshape(-1)
    rows = _sc_gather(table, idxf).reshape(_B, k, m, dpad)
    return pl.pallas_call(
        functools.partial(_sa_body_pre, k, cin),
        grid=(_B, m // tm),
        compiler_params=pltpu.CompilerParams(
            dimension_semantics=("parallel", "arbitrary")),
        in_specs=[
            pl.BlockSpec((1, k, tm, dpad), lambda b, t: (b, 0, t, 0)),
            pl.BlockSpec((1, tm, 3), lambda b, t: (b, t, 0)),
            wspec((cin, hid)), wspec((1, hid)), wspec((hid, cout)),
            wspec((1, cout)), wspec((cin, cout)), wspec((1, cout)),
        ],
        out_specs=pl.BlockSpec((1, tm, cout), lambda b, t: (b, t, 0)),
        out_shape=jax.ShapeDtypeStruct((_B, m, cout), _F32),
    )(rows, pos_dst,
      w['l1']['w'], w['l1']['b'].reshape(1, -1),
      w['l2']['w'], w['l2']['b'].reshape(1, -1),
      w['sc']['w'], w['sc']['b'].reshape(1, -1))


def _fp_body(ns, has_xt, has_sc, final, srcc_ref, srcp_ref, xs_ref, *rest):
    rest = list(rest)
    xt_ref = rest.pop(0) if has_xt else None
    w1_ref, b1_ref, w2_ref, b2_ref = rest[:4]
    rest = rest[4:]
    if has_sc:
        wsc_ref, bsc_ref = rest[:2]
        rest = rest[2:]
    if final:
        f1w_ref, f1b_ref, f2w_ref, f2b_ref = rest[:4]
        rest = rest[4:]
    pt_ref = rest[0]
    out_ref = rest[1]

    sc = srcc_ref[0]
    sx = sc[0:1, :]
    sy = sc[1:2, :]
    sz = sc[2:3, :]
    pt = pt_ref[0]
    dx = pt[:, 0:1]
    dy = pt[:, 1:2]
    dz = pt[:, 2:3]
    d2 = (dx - sx) ** 2 + (dy - sy) ** 2 + (dz - sz) ** 2
    tm = d2.shape[0]
    iota = jax.lax.broadcasted_iota(jnp.int32, (tm, ns), 1)
    parts = _split3(xs_ref[0])
    num = None
    den = None
    for _ in range(3):
        v = jnp.min(d2, axis=1, keepdims=True)
        ix = jnp.min(jnp.where(d2 == v, iota, ns), axis=1, keepdims=True)
        d2 = jnp.where(iota == ix, jnp.inf, d2)
        oh = (iota == ix).astype(_BF)
        xg = _oh_gather(oh, parts)
        wgt = 1.0 / (v + 1e-16)
        contrib = xg * wgt
        num = contrib if num is None else num + contrib
        den = wgt if den is None else den + wgt
    interp = num / den
    if has_xt:
        comb = jnp.concatenate([xt_ref[0], interp], axis=1)
    else:
        comb = interp
    h = _dot(_relu(_dot(comb, w1_ref[:]) + b1_ref[:]),
             w2_ref[:]) + b2_ref[:]
    if has_sc:
        h = h + _dot(comb, wsc_ref[:]) \
            + bsc_ref[:]
    else:
        h = h + comb
    h = _relu(h)
    if final:
        h = _relu(_dot(h, f1w_ref[:])
                  + f1b_ref[:])
        h = _dot(h, f2w_ref[:]) + f2b_ref[:]
    out_ref[0] = h


def _fp(x_tgt, pos_tgt, x_src, pos_src, w, final=None):
    ns = pos_src.shape[1]
    mt = pos_tgt.shape[1]
    tm = min(mt, 256)
    c = x_src.shape[2]
    has_xt = x_tgt is not None
    has_sc = 'sc' in w
    cin = w['l1']['w'].shape[0]
    hid = w['l1']['w'].shape[1]
    cout = w['l2']['w'].shape[1]
    srcc = jnp.transpose(pos_src, (0, 2, 1))
    wspec = lambda s: pl.BlockSpec(s, lambda b, t: tuple(0 for _ in s))
    in_specs = [
        pl.BlockSpec((1, 3, ns), lambda b, t: (b, 0, 0)),
        pl.BlockSpec((1, ns, 3), lambda b, t: (b, 0, 0)),
        pl.BlockSpec((1, ns, c), lambda b, t: (b, 0, 0)),
    ]
    args = [srcc, pos_src, x_src]
    if has_xt:
        ct = x_tgt.shape[2]
        in_specs.append(pl.BlockSpec((1, tm, ct), lambda b, t: (b, t, 0)))
        args.append(x_tgt)
    in_specs += [wspec((cin, hid)), wspec((1, hid)), wspec((hid, cout)),
                 wspec((1, cout))]
    args += [w['l1']['w'], w['l1']['b'].reshape(1, -1),
             w['l2']['w'], w['l2']['b'].reshape(1, -1)]
    if has_sc:
        in_specs += [wspec((cin, cout)), wspec((1, cout))]
        args += [w['sc']['w'], w['sc']['b'].reshape(1, -1)]
    cfin = cout
    if final is not None:
        f1, f2 = final
        h1 = f1['w'].shape[1]
        cfin = f2['w'].shape[1]
        in_specs += [wspec((cout, h1)), wspec((1, h1)),
                     wspec((h1, cfin)), wspec((1, cfin))]
        args += [f1['w'], f1['b'].reshape(1, -1),
                 f2['w'], f2['b'].reshape(1, -1)]
    in_specs.append(pl.BlockSpec((1, tm, 3), lambda b, t: (b, t, 0)))
    args.append(pos_tgt)
    return pl.pallas_call(
        functools.partial(_fp_body, ns, has_xt, has_sc, final is not None),
        grid=(_B, mt // tm),
        compiler_params=pltpu.CompilerParams(
            dimension_semantics=("parallel", "arbitrary")),
        in_specs=in_specs,
        out_specs=pl.BlockSpec((1, tm, cfin), lambda b, t: (b, t, 0)),
        out_shape=jax.ShapeDtypeStruct((_B, mt, cfin), _F32),
    )(*args)


def _head_body(x4_ref, w1_ref, b1_ref, w2_ref, b2_ref, gw_ref, gb_ref,
               out_ref):
    g = jnp.max(x4_ref[0], axis=0, keepdims=True)
    h = _dot(_relu(_dot(g, w1_ref[:]) + b1_ref[:]),
             w2_ref[:]) + b2_ref[:]
    h = _relu(h + g)
    out_ref[0] = _dot(h, gw_ref[:]) \
        + gb_ref[:]


def _head(x4, gm1, gm2):
    mper = x4.shape[1]
    d = x4.shape[2]
    cc = gm2['w'].shape[1]
    wspec = lambda s: pl.BlockSpec(s, lambda b: tuple(0 for _ in s))
    return pl.pallas_call(
        _head_body,
        grid=(_B,),
        in_specs=[
            pl.BlockSpec((1, mper, d), lambda b: (b, 0, 0)),
            wspec((d, d)), wspec((1, d)), wspec((d, d)), wspec((1, d)),
            wspec((d, cc)), wspec((1, cc)),
        ],
        out_specs=pl.BlockSpec((1, 1, cc), lambda b: (b, 0, 0)),
        out_shape=jax.ShapeDtypeStruct((_B, 1, cc), _F32),
    )(x4, gm1['l1']['w'], gm1['l1']['b'].reshape(1, -1),
      gm1['l2']['w'], gm1['l2']['b'].reshape(1, -1),
      gm2['w'], gm2['b'].reshape(1, -1))


def _bott_body(x3_ref, z_ref, c_ref, w1_ref, b1_ref, w2_ref, b2_ref,
               wsc_ref, bsc_ref, out_ref):
    x3 = x3_ref[0]
    m = x3.shape[0]
    zb = jnp.broadcast_to(z_ref[0], (m, z_ref.shape[2]))
    cb = jnp.broadcast_to(c_ref[0], (m, c_ref.shape[2]))
    comb = jnp.concatenate([x3, zb, cb], axis=1)
    h = _dot(_relu(_dot(comb, w1_ref[:]) + b1_ref[:]),
             w2_ref[:]) + b2_ref[:]
    h = h + _dot(comb, wsc_ref[:]) \
        + bsc_ref[:]
    out_ref[0] = _relu(h)


def _bott(x3, z, c, w):
    m = x3.shape[1]
    cx = x3.shape[2]
    zd = z.shape[2]
    cd = c.shape[2]
    cin = w['l1']['w'].shape[0]
    hid = w['l1']['w'].shape[1]
    cout = w['l2']['w'].shape[1]
    wspec = lambda s: pl.BlockSpec(s, lambda b: tuple(0 for _ in s))
    return pl.pallas_call(
        _bott_body,
        grid=(_B,),
        in_specs=[
            pl.BlockSpec((1, m, cx), lambda b: (b, 0, 0)),
            pl.BlockSpec((1, 1, zd), lambda b: (b, 0, 0)),
            pl.BlockSpec((1, 1, cd), lambda b: (b, 0, 0)),
            wspec((cin, hid)), wspec((1, hid)), wspec((hid, cout)),
            wspec((1, cout)), wspec((cin, cout)), wspec((1, cout)),
        ],
        out_specs=pl.BlockSpec((1, m, cout), lambda b: (b, 0, 0)),
        out_shape=jax.ShapeDtypeStruct((_B, m, cout), _F32),
    )(x3, z, c,
      w['l1']['w'], w['l1']['b'].reshape(1, -1),
      w['l2']['w'], w['l2']['b'].reshape(1, -1),
      w['sc']['w'], w['sc']['b'].reshape(1, -1))


def _sa_rest(x_src, pos, samp, w):
    pos_dst = _gather_rows(pos, samp)
    nbr, _ = _knn(pos, pos_dst, _KNN)
    xo = _sa(nbr, pos_dst, pos, x_src, w)
    return xo, pos_dst


def kernel(x, pos, batch, query_pos, query_pos_batch, params):
    cond = params['cond']
    dec = params['dec']
    pos0 = pos.reshape(_B, -1, 3)
    x0 = x.reshape(_B, -1, 3)
    q0 = query_pos.reshape(_B, -1, 3)

    se1, sd1 = _fps2(pos0, 1024, q0, 1024)
    xe1, pe1 = _sa_rest(x0, pos0, se1, cond['sa1'])
    xd1, pd1 = _sa_rest(None, q0, sd1, dec['sa1'])
    se2, sd2 = _fps2(pe1, 512, pd1, 256)
    xe2, pe2 = _sa_rest(xe1, pe1, se2, cond['sa2'])
    xd2, pd2 = _sa_rest(xd1, pd1, sd2, dec['sa2'])
    se3, sd3 = _fps2(pe2, 256, pd2, 64)
    xe3, pe3 = _sa_rest(xe2, pe2, se3, cond['sa3'])
    xd3, pd3 = _sa_rest(xd2, pd2, sd3, dec['sa3'])
    se4 = _fps(pe3, 128)
    xe4, _ = _sa_rest(xe3, pe3, se4, cond['sa4'])
    c = _head(xe4, cond['gm1'], cond['gm2'])

    z = jax.random.normal(jax.random.key(42), (_B, 64), dtype=_F32)
    bott = _bott(xd3, z.reshape(_B, 1, 64), c, dec['bott'])

    up3 = _fp(xd3, pd3, bott, pd3, dec['fp1'])
    up2 = _fp(xd2, pd2, up3, pd3, dec['fp2'])
    up1 = _fp(xd1, pd1, up2, pd2, dec['fp3'])
    out = _fp(None, q0, up1, pd1, dec['fp4'],
              final=(dec['f1'], dec['f2']))
    return out.reshape(-1, 3)
